# baseline (speedup 1.0000x reference)
import functools

import jax
import jax.numpy as jnp
from jax import lax
from jax.experimental import pallas as pl
from jax.experimental.pallas import tpu as pltpu
from jax.experimental.pallas import tpu_sc as plsc

NA = 10000
NE = 160000
NG = 320000
EMB = 64
NRBF = 6
CUT = 5.0
NBILIN = 8
NCONV = 3

NW = 32
LANES = 16

f32 = jnp.float32
i32 = jnp.int32
bf16 = jnp.bfloat16


def _mxu(a, b):
    return jnp.dot(a.astype(bf16), b.astype(bf16), preferred_element_type=f32)

R_EDGE = 16384
SHIFT_EDGE = 14
NRANGE_EDGE = -(-NE // R_EDGE)
R_ATOM = 10112
TRASH = 16


def _iota16():
    return lax.iota(i32, LANES)


def _vload(ref, off):
    return plsc.load_gather(ref, [off + _iota16()])


def _vstore(ref, off, x, mask=None):
    plsc.store_scatter(ref, [off + _iota16()], x, mask=mask)


def _tile_chunk_range(wid, n_chunks):
    q, rem = divmod(n_chunks, NW)
    count = q + jnp.where(wid < rem, 1, 0)
    first = wid * q + jnp.minimum(wid, rem)
    return first, count


def _sc_geom_body(xs_h, ys_h, zs_h, za_h, a1_h, a2_h, nb0_h, nb1_h,
                  g0_h, g1_h, g2_h,
                  dsq_h, e1_h, adot_h, acsq_h,
                  xs_v, ys_v, zs_v, za_v, a1_v, a2_v,
                  eb0, eb1, dq_b, e1_b,
                  gb0, gb1, gb2, ad_b, ac_b):
    wid = lax.axis_index("s") * 2 + lax.axis_index("c")
    pltpu.sync_copy(xs_h, xs_v)
    pltpu.sync_copy(ys_h, ys_v)
    pltpu.sync_copy(zs_h, zs_v)
    pltpu.sync_copy(za_h, za_v)
    pltpu.sync_copy(a1_h, a1_v)
    pltpu.sync_copy(a2_h, a2_v)

    CE = 400
    first, count = _tile_chunk_range(wid, NE // CE)

    def edge_chunk(c, _):
        base = (first + c) * CE
        pltpu.sync_copy(nb0_h.at[pl.ds(base, CE)], eb0)
        pltpu.sync_copy(nb1_h.at[pl.ds(base, CE)], eb1)

        def grp(g, _):
            off = g * LANES
            s = _vload(eb0, off)
            t = _vload(eb1, off)
            dx = plsc.load_gather(xs_v, [s]) - plsc.load_gather(xs_v, [t])
            dy = plsc.load_gather(ys_v, [s]) - plsc.load_gather(ys_v, [t])
            dz = plsc.load_gather(zs_v, [s]) - plsc.load_gather(zs_v, [t])
            _vstore(dq_b, off, dx * dx + dy * dy + dz * dz)
            zi = plsc.load_gather(za_v, [s]) * EMB
            zj = plsc.load_gather(za_v, [t]) * EMB
            eoff = off * EMB + _iota16() * EMB
            for ccol in range(EMB):
                v = (plsc.load_gather(a1_v, [zi + ccol]) +
                     plsc.load_gather(a2_v, [zj + ccol]))
                plsc.store_scatter(e1_b, [eoff + ccol], v)
            return 0

        lax.fori_loop(0, CE // LANES, grp, 0)
        pltpu.sync_copy(dq_b, dsq_h.at[pl.ds(base, CE)])
        pltpu.sync_copy(e1_b, e1_h.at[pl.ds(base * EMB, CE * EMB)])
        return 0

    lax.fori_loop(0, count, edge_chunk, 0)

    CA = 512
    afirst, acount = _tile_chunk_range(wid, NG // CA)

    def ang_chunk(c, _):
        base = (afirst + c) * CA
        pltpu.sync_copy(g0_h.at[pl.ds(base, CA)], gb0)
        pltpu.sync_copy(g1_h.at[pl.ds(base, CA)], gb1)
        pltpu.sync_copy(g2_h.at[pl.ds(base, CA)], gb2)

        def grp(g, _):
            off = g * LANES
            ia = _vload(gb0, off)
            ib = _vload(gb1, off)
            ic = _vload(gb2, off)
            bx = plsc.load_gather(xs_v, [ib])
            by = plsc.load_gather(ys_v, [ib])
            bz = plsc.load_gather(zs_v, [ib])
            jx = plsc.load_gather(xs_v, [ia]) - bx
            jy = plsc.load_gather(ys_v, [ia]) - by
            jz = plsc.load_gather(zs_v, [ia]) - bz
            kx = plsc.load_gather(xs_v, [ic]) - bx
            ky = plsc.load_gather(ys_v, [ic]) - by
            kz = plsc.load_gather(zs_v, [ic]) - bz
            _vstore(ad_b, off, jx * kx + jy * ky + jz * kz)
            cx = jy * kz - jz * ky
            cy = jz * kx - jx * kz
            cz = jx * ky - jy * kx
            _vstore(ac_b, off, cx * cx + cy * cy + cz * cz)
            return 0

        lax.fori_loop(0, CA // LANES, grp, 0)
        pltpu.sync_copy(ad_b, adot_h.at[pl.ds(base, CA)])
        pltpu.sync_copy(ac_b, acsq_h.at[pl.ds(base, CA)])
        return 0

    lax.fori_loop(0, acount, ang_chunk, 0)


def _sc_geom(xs, ys, zs, za, a1f, a2f, nb0, nb1, g0, g1, g2):
    CE, CA = 400, 512
    kern = pl.kernel(
        _sc_geom_body,
        out_type=(
            jax.ShapeDtypeStruct((NE,), f32),
            jax.ShapeDtypeStruct((NE * EMB,), f32),
            jax.ShapeDtypeStruct((NG,), f32),
            jax.ShapeDtypeStruct((NG,), f32),
        ),
        mesh=plsc.VectorSubcoreMesh(core_axis_name="c", subcore_axis_name="s"),
        compiler_params=pltpu.CompilerParams(needs_layout_passes=False, use_tc_tiling_on_sc=False),
        scratch_types=[
            pltpu.VMEM((NA,), f32), pltpu.VMEM((NA,), f32),
            pltpu.VMEM((NA,), f32), pltpu.VMEM((NA,), i32),
            pltpu.VMEM((95 * EMB,), f32), pltpu.VMEM((95 * EMB,), f32),
            pltpu.VMEM((CE,), i32), pltpu.VMEM((CE,), i32),
            pltpu.VMEM((CE,), f32), pltpu.VMEM((CE * EMB,), f32),
            pltpu.VMEM((CA,), i32), pltpu.VMEM((CA,), i32),
            pltpu.VMEM((CA,), i32),
            pltpu.VMEM((CA,), f32), pltpu.VMEM((CA,), f32),
        ],
    )
    return kern(xs, ys, zs, za, a1f, a2f, nb0, nb1, g0, g1, g2)


def _make_gather(width):
    SCR = 512
    n_contrib = NG // NW

    def body(table_h, idx_h, out_h, idx_v, vb0, vb1, semg0, semg1):
        wid = lax.axis_index("s") * 2 + lax.axis_index("c")
        tbase = wid * n_contrib
        pltpu.sync_copy(idx_h.at[pl.ds(tbase, n_contrib)], idx_v)
        chunks = _static_chunks(n_contrib, SCR)
        vbufs = (vb0, vb1)
        sems_g = (semg0, semg1)
        dload = {}

        def start_load(ci):
            off, sz = chunks[ci]
            dload[ci] = [pltpu.async_copy(
                table_h.at[idx_v.at[pl.ds(off + koff, ksz)]],
                vbufs[ci % 2].at[pl.ds(koff, ksz)], sems_g[ci % 2])
                for koff, ksz in _static_chunks(sz, 128)]

        start_load(0)
        for ci, (off, sz) in enumerate(chunks):
            if ci + 1 < len(chunks):
                start_load(ci + 1)
            for d in dload.pop(ci):
                d.wait()
            pltpu.sync_copy(vbufs[ci % 2].at[pl.ds(0, sz)],
                            out_h.at[pl.ds(tbase + off, sz)])

    out_shape = (NG, EMB) if width else (NG,)
    buf_shape = (SCR, EMB) if width else (SCR,)
    dt = bf16 if width else f32
    kern = pl.kernel(
        body,
        out_type=jax.ShapeDtypeStruct(out_shape, dt),
        mesh=plsc.VectorSubcoreMesh(core_axis_name="c", subcore_axis_name="s"),
        compiler_params=pltpu.CompilerParams(needs_layout_passes=False,
                                             use_tc_tiling_on_sc=False),
        scratch_types=[
            pltpu.VMEM((n_contrib,), i32),
            pltpu.VMEM(buf_shape, dt),
            pltpu.VMEM(buf_shape, dt),
            pltpu.SemaphoreType.DMA,
            pltpu.SemaphoreType.DMA,
        ],
    )
    return kern


_gather_rows_k = None
_gather_scalar_k = None


def _sc_gather_rows(table, idx):
    global _gather_rows_k
    if _gather_rows_k is None:
        _gather_rows_k = _make_gather(EMB)
    return _gather_rows_k(table, idx)


def _sc_gather_scalar(table, idx):
    global _gather_scalar_k
    if _gather_scalar_k is None:
        _gather_scalar_k = _make_gather(0)
    return _gather_scalar_k(table, idx)


def _static_chunks(total, c):
    out_list = []
    off = 0
    while off < total:
        sz = min(c, total - off)
        out_list.append((off, sz))
        off += sz
    return out_list


def _make_segsum(n_in, rng_size, n_ranges, shift):
    SCR = 512
    n_contrib = n_in // NW
    rows_per_tile = rng_size // LANES
    m_out = rng_size * n_ranges

    def body(vals_h, idx_h, out_h, idx_v, bkt_v, tgt_v, vb0, vb1, zbuf_v,
             acc_s, semg0, semg1, sems):
        cid = lax.axis_index("c")
        sid = lax.axis_index("s")
        wid = sid * 2 + cid
        tbase = wid * n_contrib
        pltpu.sync_copy(idx_h.at[pl.ds(tbase, n_contrib)],
                        idx_v.at[pl.ds(0, n_contrib)])

        def zinit(i, _):
            zbuf_v[i // 4, pl.ds((i % 4) * LANES, LANES)] = (
                jnp.zeros((LANES,), f32))
            return 0

        lax.fori_loop(0, 128 * 4, zinit, 0)

        def zero_slice():
            for (zoff, zsz) in _static_chunks(rows_per_tile, 128):
                pltpu.sync_copy(
                    zbuf_v.at[pl.ds(0, zsz)],
                    acc_s.at[pl.ds(sid * rows_per_tile + zoff, zsz)])

        def dump_slice(lo):
            pltpu.sync_copy(
                acc_s.at[pl.ds(sid * rows_per_tile, rows_per_tile)],
                out_h.at[cid].at[pl.ds(lo + sid * rows_per_tile,
                                       rows_per_tile)])

        def put_tgt(g, vals16):
            plsc.store_scatter(
                tgt_v, [g // 8 + 0 * _iota16(),
                        (g % 8) * LANES + _iota16()], vals16)

        def fire_scatters(vb):
            ds_ = [pltpu.async_copy(vb.at[pl.ds(k * 128, 128)],
                                    acc_s.at[tgt_v.at[k]], sems, add=True)
                   for k in range(4)]
            for d in ds_:
                d.wait()

        if n_ranges == 1:
            zero_slice()
            plsc.subcore_barrier()
            chunks = _static_chunks(n_contrib, SCR)
            vbufs = (vb0, vb1)
            sems_g = (semg0, semg1)
            dload = {}

            def start_load(ci):
                off, sz = chunks[ci]
                dload[ci] = pltpu.async_copy(
                    vals_h.at[pl.ds(tbase + off, sz)],
                    vbufs[ci % 2].at[pl.ds(0, sz)], sems_g[ci % 2])

            start_load(0)
            for ci, (off, sz) in enumerate(chunks):
                if ci + 1 < len(chunks):
                    start_load(ci + 1)
                dload.pop(ci).wait()
                ng = -(-sz // LANES)
                for g in range(ng):
                    iv = _vload(idx_v, off + g * LANES)
                    rem = sz - g * LANES
                    if rem < LANES:
                        iv = jnp.where(_iota16() < rem, iv, rng_size)
                    put_tgt(g, iv)
                for g in range(ng, SCR // LANES):
                    put_tgt(g, rng_size + 0 * _iota16())
                fire_scatters(vbufs[ci % 2])
            plsc.subcore_barrier()
            dump_slice(0)
        else:
            for r in range(n_ranges):
                zero_slice()
                lo = r * rng_size

                def scan_g(g, cnt):
                    iv = _vload(idx_v, g * LANES)
                    mask = lax.shift_right_logical(iv, shift) == r
                    plsc.store_compressed(
                        bkt_v.at[pl.ds(cnt, LANES)],
                        tbase + g * LANES + _iota16(), mask=mask)
                    return cnt + jnp.sum(mask.astype(i32))

                cnt = lax.fori_loop(0, n_contrib // LANES, scan_g, 0)
                for g in range(SCR // LANES):
                    _vstore(bkt_v, cnt + g * LANES, tbase + 0 * _iota16())
                plsc.subcore_barrier()

                def sc_loop(t, _):
                    soff = t * SCR
                    dg = [pltpu.async_copy(
                        vals_h.at[bkt_v.at[pl.ds(soff + k * 128, 128)]],
                        vb0.at[pl.ds(k * 128, 128)], semg0)
                        for k in range(4)]
                    for d in dg:
                        d.wait()

                    def tgt_g(g, _):
                        gpos = _vload(bkt_v, soff + g * LANES)
                        iv = plsc.load_gather(idx_v, [gpos - tbase])
                        valid = (soff + g * LANES + _iota16()) < cnt
                        put_tgt(g, jnp.where(valid, iv - lo, rng_size))
                        return 0

                    lax.fori_loop(0, SCR // LANES, tgt_g, 0)
                    fire_scatters(vb0)
                    return 0

                trips = lax.shift_right_logical(cnt + (SCR - 1), 9)
                lax.fori_loop(0, trips, sc_loop, 0)
                plsc.subcore_barrier()
                dump_slice(lo)
                plsc.subcore_barrier()

    kern = pl.kernel(
        body,
        out_type=jax.ShapeDtypeStruct((2, m_out, EMB), f32),
        mesh=plsc.VectorSubcoreMesh(core_axis_name="c", subcore_axis_name="s"),
        compiler_params=pltpu.CompilerParams(needs_layout_passes=False,
                                             use_tc_tiling_on_sc=False),
        scratch_types=[
            pltpu.VMEM((n_contrib + 16,), i32),
            pltpu.VMEM((n_contrib + SCR,), i32),
            pltpu.VMEM((4, 128), i32),
            pltpu.VMEM((SCR, EMB), f32),
            pltpu.VMEM((SCR, EMB), f32),
            pltpu.VMEM((128, EMB), f32),
            pltpu.VMEM_SHARED((rng_size + TRASH, EMB), f32),
            pltpu.SemaphoreType.DMA,
            pltpu.SemaphoreType.DMA,
            pltpu.SemaphoreType.DMA,
        ],
    )
    return kern


_segsum_edge_k = None
_segsum_atom_k = None


def _get_segsum_edge():
    global _segsum_edge_k
    if _segsum_edge_k is None:
        _segsum_edge_k = _make_segsum(NG, R_EDGE, NRANGE_EDGE, SHIFT_EDGE)
    return _segsum_edge_k


def _get_segsum_atom():
    global _segsum_atom_k
    if _segsum_atom_k is None:
        _segsum_atom_k = _make_segsum(NE, R_ATOM, 1, 0)
    return _segsum_atom_k


def _swish(x):
    return x / (1.0 + jnp.exp(-x))


def _envelope(x):
    p = 6
    a = -(p + 1) * (p + 2) / 2.0
    b = float(p * (p + 2))
    c = -p * (p + 1) / 2.0
    x2 = x * x
    x4 = x2 * x2
    x5 = x4 * x
    env = 1.0 / x + a * x5 + b * x5 * x + c * x5 * x2
    return jnp.where(x < 1.0, env, 0.0)


def _radial8(dsq):
    d = jnp.sqrt(dsq + 1e-12)
    x = d / CUT
    ni = lax.broadcasted_iota(i32, (1, 8), 1) + 1
    n = jnp.where(ni <= 6, ni, 0).astype(f32)
    return _envelope(x) * jnp.sin(n * jnp.pi * x)


def _tc_edge_init_body(dsq_ref, e1_ref, werc_ref, b_ref, wr0_ref,
                       m_ref, g0_ref, er_ref):
    er = _radial8(dsq_ref[...]) * jnp.sqrt(2.0 / CUT)
    m = _swish(e1_ref[...] + _mxu(er, werc_ref[...]) + b_ref[...])
    m_ref[...] = m
    g0_ref[...] = m * _mxu(er, wr0_ref[...])
    er_ref[...] = er


def _tc_edge_init(dsq, e1, werc8, bvec, wrbf0):
    BE = 2000
    return pl.pallas_call(
        _tc_edge_init_body,
        grid=(NE // BE,),
        in_specs=[
            pl.BlockSpec((BE, 1), lambda i: (i, 0)),
            pl.BlockSpec((BE, EMB), lambda i: (i, 0)),
            pl.BlockSpec((8, EMB), lambda i: (0, 0)),
            pl.BlockSpec((1, EMB), lambda i: (0, 0)),
            pl.BlockSpec((8, EMB), lambda i: (0, 0)),
        ],
        out_specs=[
            pl.BlockSpec((BE, EMB), lambda i: (i, 0)),
            pl.BlockSpec((BE, EMB), lambda i: (i, 0)),
            pl.BlockSpec((BE, 8), lambda i: (i, 0)),
        ],
        out_shape=[
            jax.ShapeDtypeStruct((NE, EMB), f32),
            jax.ShapeDtypeStruct((NE, EMB), f32),
            jax.ShapeDtypeStruct((NE, 8), f32),
        ],
    )(dsq, e1, werc8, bvec, wrbf0)


def _tc_sbf_body(dq_ref, ad_ref, ac_ref, wsp_ref, sp_ref):
    rad = _radial8(dq_ref[...])
    adot = ad_ref[...]
    acsq = ac_ref[...]
    cosa = adot * lax.rsqrt(adot * adot + acsq + 1e-12)
    ts = [jnp.ones_like(cosa), cosa]
    for _ in range(5):
        ts.append(2.0 * cosa * ts[-1] - ts[-2])
    ang = jnp.concatenate(ts + [jnp.zeros_like(cosa)], axis=1)
    sbf = jnp.concatenate([rad[:, n:n + 1] * ang for n in range(8)], axis=1)
    w = wsp_ref[...]
    sp0_ref, sp1_ref, sp2_ref = sp_ref
    sp0_ref[...] = jnp.dot(sbf, w[:, 0:8], preferred_element_type=f32)
    sp1_ref[...] = jnp.dot(sbf, w[:, 8:16], preferred_element_type=f32)
    sp2_ref[...] = jnp.dot(sbf, w[:, 16:24], preferred_element_type=f32)


def _tc_sbf_wrap(dq_ref, ad_ref, ac_ref, wsp_ref, s0, s1, s2):
    _tc_sbf_body(dq_ref, ad_ref, ac_ref, wsp_ref, (s0, s1, s2))


def _tc_sbf(dsq_kj, adot, acsq, wsp64):
    BA = 2000
    return pl.pallas_call(
        _tc_sbf_wrap,
        grid=(NG // BA,),
        in_specs=[
            pl.BlockSpec((BA, 1), lambda i: (i, 0)),
            pl.BlockSpec((BA, 1), lambda i: (i, 0)),
            pl.BlockSpec((BA, 1), lambda i: (i, 0)),
            pl.BlockSpec((64, 24), lambda i: (0, 0)),
        ],
        out_specs=[pl.BlockSpec((BA, 8), lambda i: (i, 0))] * 3,
        out_shape=[jax.ShapeDtypeStruct((NG, 8), f32)] * 3,
    )(dsq_kj, adot, acsq, wsp64)


def _tc_layer_pre_body(m_ref, er_ref, w1_ref, b1_ref, w2_ref, b2_ref,
                       wr_ref, xji_ref, q_ref):
    m = m_ref[...]
    er = er_ref[...]
    xji_ref[...] = _swish(_mxu(m, w1_ref[...]) + b1_ref[...])
    q_ref[...] = (_swish(_mxu(m, w2_ref[...]) + b2_ref[...])
                  * _mxu(er, wr_ref[...])).astype(bf16)


def _tc_layer_pre(m, er, w1, b1, w2, b2, wrbf8):
    BE = 2000
    return pl.pallas_call(
        _tc_layer_pre_body,
        grid=(NE // BE,),
        in_specs=[
            pl.BlockSpec((BE, EMB), lambda i: (i, 0)),
            pl.BlockSpec((BE, 8), lambda i: (i, 0)),
            pl.BlockSpec((EMB, EMB), lambda i: (0, 0)),
            pl.BlockSpec((1, EMB), lambda i: (0, 0)),
            pl.BlockSpec((EMB, EMB), lambda i: (0, 0)),
            pl.BlockSpec((1, EMB), lambda i: (0, 0)),
            pl.BlockSpec((8, EMB), lambda i: (0, 0)),
        ],
        out_specs=[
            pl.BlockSpec((BE, EMB), lambda i: (i, 0)),
            pl.BlockSpec((BE, EMB), lambda i: (i, 0)),
        ],
        out_shape=[
            jax.ShapeDtypeStruct((NE, EMB), f32),
            jax.ShapeDtypeStruct((NE, EMB), bf16),
        ],
    )(m, er, w1, b1, w2, b2, wrbf8)


def _tc_bilinear_body(qk_ref, sp_ref, wb_ref, y_ref):
    h = _mxu(qk_ref[...], wb_ref[...])
    sp = sp_ref[...]
    y = sp[:, 0:1] * h[:, 0:EMB]
    for b in range(1, NBILIN):
        y = y + sp[:, b:b + 1] * h[:, b * EMB:(b + 1) * EMB]
    y_ref[...] = y


def _tc_bilinear(qk, spl, wbcat):
    BA = 4000
    return pl.pallas_call(
        _tc_bilinear_body,
        grid=(NG // BA,),
        in_specs=[
            pl.BlockSpec((BA, EMB), lambda i: (i, 0)),
            pl.BlockSpec((BA, 8), lambda i: (i, 0)),
            pl.BlockSpec((EMB, EMB * NBILIN), lambda i: (0, 0)),
        ],
        out_specs=pl.BlockSpec((BA, EMB), lambda i: (i, 0)),
        out_shape=jax.ShapeDtypeStruct((NG, EMB), f32),
    )(qk, spl, wbcat)


def _tc_layer_post_body(m_ref, xji_ref, a0_ref, a1_ref, er_ref,
                        wres_ref, bres_ref, wr_ref, mn_ref, g_ref):
    u = xji_ref[...] + a0_ref[0] + a1_ref[0]
    mn = m_ref[...] + _swish(_mxu(u, wres_ref[...]) + bres_ref[...])
    mn_ref[...] = mn
    g_ref[...] = mn * _mxu(er_ref[...], wr_ref[...])


def _tc_layer_post(m, xji, aggp, er, wres, bres, wrbf8):
    BE = 2000
    return pl.pallas_call(
        _tc_layer_post_body,
        grid=(NE // BE,),
        in_specs=[
            pl.BlockSpec((BE, EMB), lambda i: (i, 0)),
            pl.BlockSpec((BE, EMB), lambda i: (i, 0)),
            pl.BlockSpec((1, BE, EMB), lambda i: (0, i, 0)),
            pl.BlockSpec((1, BE, EMB), lambda i: (1, i, 0)),
            pl.BlockSpec((BE, 8), lambda i: (i, 0)),
            pl.BlockSpec((EMB, EMB), lambda i: (0, 0)),
            pl.BlockSpec((1, EMB), lambda i: (0, 0)),
            pl.BlockSpec((8, EMB), lambda i: (0, 0)),
        ],
        out_specs=[
            pl.BlockSpec((BE, EMB), lambda i: (i, 0)),
            pl.BlockSpec((BE, EMB), lambda i: (i, 0)),
        ],
        out_shape=[
            jax.ShapeDtypeStruct((NE, EMB), f32),
            jax.ShapeDtypeStruct((NE, EMB), f32),
        ],
    )(m, xji, aggp, aggp, er, wres, bres, wrbf8)


def _tc_atom_body(*refs):
    i = pl.program_id(0)
    out_ref = refs[-1]
    BT = refs[0].shape[1]
    rows = i * BT + lax.broadcasted_iota(i32, (BT, 1), 0)
    valid = (rows < NA).astype(f32)
    total = jnp.zeros((), f32)
    for k in range(NCONV + 1):
        tp0 = refs[2 * k][0]
        tp1 = refs[2 * k + 1][0]
        wh = refs[8 + 3 * k][...]
        bh = refs[8 + 3 * k + 1][...]
        wo = refs[8 + 3 * k + 2][...]
        s = _swish(_mxu(tp0 + tp1, wh) + bh)
        contrib = _mxu(s, wo) * valid
        total = total + jnp.sum(contrib)

    @pl.when(i == 0)
    def _():
        out_ref[...] = jnp.zeros((1, 1), f32)

    out_ref[...] += jnp.reshape(total, (1, 1))


def _tc_atom(tps, wsets):
    BT = 2528
    in_specs = []
    args = []
    for tp in tps:
        in_specs += [pl.BlockSpec((1, BT, EMB), lambda i: (0, i, 0)),
                     pl.BlockSpec((1, BT, EMB), lambda i: (1, i, 0))]
        args += [tp, tp]
    for (wh, bh, wo) in wsets:
        in_specs += [pl.BlockSpec((EMB, EMB), lambda i: (0, 0)),
                     pl.BlockSpec((1, EMB), lambda i: (0, 0)),
                     pl.BlockSpec((EMB, 1), lambda i: (0, 0))]
        args += [wh, bh, wo]
    return pl.pallas_call(
        _tc_atom_body,
        grid=(R_ATOM // BT,),
        in_specs=in_specs,
        out_specs=pl.BlockSpec((1, 1), lambda i: (0, 0)),
        out_shape=jax.ShapeDtypeStruct((1, 1), f32),
    )(*args)


def kernel(nxyz, params, nbr_list, angle_list, num_atoms, ji_idx, kj_idx):
    xs = jnp.asarray(nxyz[:, 1], f32)
    ys = jnp.asarray(nxyz[:, 2], f32)
    zs = jnp.asarray(nxyz[:, 3], f32)
    za = nxyz[:, 0].astype(i32)
    nb0 = jnp.asarray(nbr_list[:, 0], i32)
    nb1 = jnp.asarray(nbr_list[:, 1], i32)
    g0 = jnp.asarray(angle_list[:, 0], i32)
    g1 = jnp.asarray(angle_list[:, 1], i32)
    g2 = jnp.asarray(angle_list[:, 2], i32)
    ji = jnp.asarray(ji_idx, i32)
    kj = jnp.asarray(kj_idx, i32)

    W = params["emb_W"]
    a1f = (params["emb"] @ W[0:EMB]).reshape(-1)
    a2f = (params["emb"] @ W[EMB:2 * EMB]).reshape(-1)
    werc8 = jnp.pad(params["emb_Wrbf"] @ W[2 * EMB:], ((0, 2), (0, 0)))
    bvec = params["emb_b"].reshape(1, EMB)

    def pad8(w):
        return jnp.pad(w, ((0, 2), (0, 0)))

    wsp_list = []
    wb_list = []
    for l in range(NCONV):
        p = params["int"][l]
        wsp_list.append(
            jnp.pad(p["W_sbf"].reshape(6, 7, NBILIN),
                    ((0, 2), (0, 1), (0, 0))).reshape(64, NBILIN))
        wb_list.append(jnp.transpose(p["W_bilin"], (1, 0, 2))
                       .reshape(EMB, NBILIN * EMB))
    wsp64 = jnp.concatenate(wsp_list, axis=1)

    dsq, e1f, adot, acsq = _sc_geom(xs, ys, zs, za, a1f, a2f,
                                    nb0, nb1, g0, g1, g2)
    e1 = e1f.reshape(NE, EMB)
    dsq2 = dsq.reshape(NE, 1)

    m, g0e, er = _tc_edge_init(dsq2, e1, werc8, bvec,
                               pad8(params["out"][0]["W_rbf"]))

    dsq_kj = _sc_gather_scalar(dsq, kj).reshape(NG, 1)
    sp = _tc_sbf(dsq_kj, adot.reshape(NG, 1), acsq.reshape(NG, 1), wsp64)

    segsum_atom = _get_segsum_atom()
    segsum_edge = _get_segsum_edge()
    tps = [segsum_atom(g0e, nb0)]

    for l in range(NCONV):
        p = params["int"][l]
        xji, q = _tc_layer_pre(m, er, p["W1"], p["b1"].reshape(1, EMB),
                               p["W2"], p["b2"].reshape(1, EMB),
                               pad8(p["W_rbf"]))
        qk = _sc_gather_rows(q, kj)
        y = _tc_bilinear(qk, sp[l], wb_list[l])
        aggp = segsum_edge(y, ji)
        m, ge = _tc_layer_post(m, xji, aggp, er,
                               p["W_res"], p["b_res"].reshape(1, EMB),
                               pad8(params["out"][l + 1]["W_rbf"]))
        tps.append(segsum_atom(ge, nb0))

    wsets = [(po["W_h"], po["b_h"].reshape(1, EMB), po["W_out"])
             for po in params["out"]]
    total = _tc_atom(tps, wsets)
    return jnp.reshape(total, (1,))


# speedup vs baseline: 1.4157x; 1.4157x over previous
import functools

import jax
import jax.numpy as jnp
from jax import lax
from jax.experimental import pallas as pl
from jax.experimental.pallas import tpu as pltpu
from jax.experimental.pallas import tpu_sc as plsc

NA = 10000
NE = 160000
NG = 320000
EMB = 64
NRBF = 6
CUT = 5.0
NBILIN = 8
NCONV = 3

NW = 32
LANES = 16

f32 = jnp.float32
i32 = jnp.int32
bf16 = jnp.bfloat16


def _mxu(a, b):
    return jnp.dot(a.astype(bf16), b.astype(bf16), preferred_element_type=f32)

R_EDGE = 16384
SHIFT_EDGE = 14
NRANGE_EDGE = -(-NE // R_EDGE)
R_ATOM = 10112
TRASH = 16


def _iota16():
    return lax.iota(i32, LANES)


def _vload(ref, off):
    return plsc.load_gather(ref, [off + _iota16()])


def _vstore(ref, off, x, mask=None):
    plsc.store_scatter(ref, [off + _iota16()], x, mask=mask)


def _tile_chunk_range(wid, n_chunks):
    q, rem = divmod(n_chunks, NW)
    count = q + jnp.where(wid < rem, 1, 0)
    first = wid * q + jnp.minimum(wid, rem)
    return first, count


def _sc_geom_body(xs_h, ys_h, zs_h, za_h, a1_h, a2_h, nb0_h, nb1_h,
                  g0_h, g1_h, g2_h,
                  dsq_h, e1_h, adot_h, acsq_h,
                  xs_v, ys_v, zs_v, za_v, a1_v, a2_v,
                  eb0, eb1, dq_b, e1_b,
                  gb0, gb1, gb2, ad_b, ac_b):
    wid = lax.axis_index("s") * 2 + lax.axis_index("c")
    pltpu.sync_copy(xs_h, xs_v)
    pltpu.sync_copy(ys_h, ys_v)
    pltpu.sync_copy(zs_h, zs_v)
    pltpu.sync_copy(za_h, za_v)
    pltpu.sync_copy(a1_h, a1_v)
    pltpu.sync_copy(a2_h, a2_v)

    CE = 400
    first, count = _tile_chunk_range(wid, NE // CE)

    def edge_chunk(c, _):
        base = (first + c) * CE
        pltpu.sync_copy(nb0_h.at[pl.ds(base, CE)], eb0)
        pltpu.sync_copy(nb1_h.at[pl.ds(base, CE)], eb1)

        def grp(g, _):
            off = g * LANES
            s = _vload(eb0, off)
            t = _vload(eb1, off)
            dx = plsc.load_gather(xs_v, [s]) - plsc.load_gather(xs_v, [t])
            dy = plsc.load_gather(ys_v, [s]) - plsc.load_gather(ys_v, [t])
            dz = plsc.load_gather(zs_v, [s]) - plsc.load_gather(zs_v, [t])
            _vstore(dq_b, off, dx * dx + dy * dy + dz * dz)
            zi = plsc.load_gather(za_v, [s]) * EMB
            zj = plsc.load_gather(za_v, [t]) * EMB
            eoff = off * EMB + _iota16() * EMB
            for ccol in range(EMB):
                v = (plsc.load_gather(a1_v, [zi + ccol]) +
                     plsc.load_gather(a2_v, [zj + ccol]))
                plsc.store_scatter(e1_b, [eoff + ccol], v)
            return 0

        lax.fori_loop(0, CE // LANES, grp, 0)
        pltpu.sync_copy(dq_b, dsq_h.at[pl.ds(base, CE)])
        pltpu.sync_copy(e1_b, e1_h.at[pl.ds(base * EMB, CE * EMB)])
        return 0

    lax.fori_loop(0, count, edge_chunk, 0)

    CA = 512
    afirst, acount = _tile_chunk_range(wid, NG // CA)

    def ang_chunk(c, _):
        base = (afirst + c) * CA
        pltpu.sync_copy(g0_h.at[pl.ds(base, CA)], gb0)
        pltpu.sync_copy(g1_h.at[pl.ds(base, CA)], gb1)
        pltpu.sync_copy(g2_h.at[pl.ds(base, CA)], gb2)

        def grp(g, _):
            off = g * LANES
            ia = _vload(gb0, off)
            ib = _vload(gb1, off)
            ic = _vload(gb2, off)
            bx = plsc.load_gather(xs_v, [ib])
            by = plsc.load_gather(ys_v, [ib])
            bz = plsc.load_gather(zs_v, [ib])
            jx = plsc.load_gather(xs_v, [ia]) - bx
            jy = plsc.load_gather(ys_v, [ia]) - by
            jz = plsc.load_gather(zs_v, [ia]) - bz
            kx = plsc.load_gather(xs_v, [ic]) - bx
            ky = plsc.load_gather(ys_v, [ic]) - by
            kz = plsc.load_gather(zs_v, [ic]) - bz
            _vstore(ad_b, off, jx * kx + jy * ky + jz * kz)
            cx = jy * kz - jz * ky
            cy = jz * kx - jx * kz
            cz = jx * ky - jy * kx
            _vstore(ac_b, off, cx * cx + cy * cy + cz * cz)
            return 0

        lax.fori_loop(0, CA // LANES, grp, 0)
        pltpu.sync_copy(ad_b, adot_h.at[pl.ds(base, CA)])
        pltpu.sync_copy(ac_b, acsq_h.at[pl.ds(base, CA)])
        return 0

    lax.fori_loop(0, acount, ang_chunk, 0)


def _sc_geom(xs, ys, zs, za, a1f, a2f, nb0, nb1, g0, g1, g2):
    CE, CA = 400, 512
    kern = pl.kernel(
        _sc_geom_body,
        out_type=(
            jax.ShapeDtypeStruct((NE,), f32),
            jax.ShapeDtypeStruct((NE * EMB,), f32),
            jax.ShapeDtypeStruct((NG,), f32),
            jax.ShapeDtypeStruct((NG,), f32),
        ),
        mesh=plsc.VectorSubcoreMesh(core_axis_name="c", subcore_axis_name="s"),
        compiler_params=pltpu.CompilerParams(needs_layout_passes=False, use_tc_tiling_on_sc=False),
        scratch_types=[
            pltpu.VMEM((NA,), f32), pltpu.VMEM((NA,), f32),
            pltpu.VMEM((NA,), f32), pltpu.VMEM((NA,), i32),
            pltpu.VMEM((95 * EMB,), f32), pltpu.VMEM((95 * EMB,), f32),
            pltpu.VMEM((CE,), i32), pltpu.VMEM((CE,), i32),
            pltpu.VMEM((CE,), f32), pltpu.VMEM((CE * EMB,), f32),
            pltpu.VMEM((CA,), i32), pltpu.VMEM((CA,), i32),
            pltpu.VMEM((CA,), i32),
            pltpu.VMEM((CA,), f32), pltpu.VMEM((CA,), f32),
        ],
    )
    return kern(xs, ys, zs, za, a1f, a2f, nb0, nb1, g0, g1, g2)


def _make_gather(width):
    SCR = 512
    n_contrib = NG // NW

    def body(table_h, idx_h, out_h, idx_v, vb0, vb1, semg0, semg1):
        wid = lax.axis_index("s") * 2 + lax.axis_index("c")
        tbase = wid * n_contrib
        pltpu.sync_copy(idx_h.at[pl.ds(tbase, n_contrib)], idx_v)
        chunks = _static_chunks(n_contrib, SCR)
        vbufs = (vb0, vb1)
        sems_g = (semg0, semg1)
        dload = {}

        def start_load(ci):
            off, sz = chunks[ci]
            dload[ci] = [pltpu.async_copy(
                table_h.at[idx_v.at[pl.ds(off + koff, ksz)]],
                vbufs[ci % 2].at[pl.ds(koff, ksz)], sems_g[ci % 2])
                for koff, ksz in _static_chunks(sz, 128)]

        start_load(0)
        for ci, (off, sz) in enumerate(chunks):
            if ci + 1 < len(chunks):
                start_load(ci + 1)
            for d in dload.pop(ci):
                d.wait()
            pltpu.sync_copy(vbufs[ci % 2].at[pl.ds(0, sz)],
                            out_h.at[pl.ds(tbase + off, sz)])

    out_shape = (NG, EMB) if width else (NG,)
    buf_shape = (SCR, EMB) if width else (SCR,)
    dt = bf16 if width else f32
    kern = pl.kernel(
        body,
        out_type=jax.ShapeDtypeStruct(out_shape, dt),
        mesh=plsc.VectorSubcoreMesh(core_axis_name="c", subcore_axis_name="s"),
        compiler_params=pltpu.CompilerParams(needs_layout_passes=False,
                                             use_tc_tiling_on_sc=False),
        scratch_types=[
            pltpu.VMEM((n_contrib,), i32),
            pltpu.VMEM(buf_shape, dt),
            pltpu.VMEM(buf_shape, dt),
            pltpu.SemaphoreType.DMA,
            pltpu.SemaphoreType.DMA,
        ],
    )
    return kern


_gather_rows_k = None
_gather_scalar_k = None


def _sc_gather_rows(table, idx):
    global _gather_rows_k
    if _gather_rows_k is None:
        _gather_rows_k = _make_gather(EMB)
    return _gather_rows_k(table, idx)


def _sc_gather_scalar(table, idx):
    global _gather_scalar_k
    if _gather_scalar_k is None:
        _gather_scalar_k = _make_gather(0)
    return _gather_scalar_k(table, idx)


def _static_chunks(total, c):
    out_list = []
    off = 0
    while off < total:
        sz = min(c, total - off)
        out_list.append((off, sz))
        off += sz
    return out_list


def _make_segsum(n_in, rng_size, n_ranges, shift):
    SCR = 512
    n_contrib = n_in // NW
    rows_per_tile = rng_size // LANES
    m_out = rng_size * n_ranges

    def body(vals_h, idx_h, out_h, idx_v, bkt_v, tgt_v, vb0, vb1, zbuf_v,
             acc_s, semg0, semg1, sems):
        cid = lax.axis_index("c")
        sid = lax.axis_index("s")
        wid = sid * 2 + cid
        tbase = wid * n_contrib
        pltpu.sync_copy(idx_h.at[pl.ds(tbase, n_contrib)],
                        idx_v.at[pl.ds(0, n_contrib)])

        def zinit(i, _):
            zbuf_v[i // 4, pl.ds((i % 4) * LANES, LANES)] = (
                jnp.zeros((LANES,), f32))
            return 0

        lax.fori_loop(0, 128 * 4, zinit, 0)

        def zero_slice():
            for (zoff, zsz) in _static_chunks(rows_per_tile, 128):
                pltpu.sync_copy(
                    zbuf_v.at[pl.ds(0, zsz)],
                    acc_s.at[pl.ds(sid * rows_per_tile + zoff, zsz)])

        def dump_slice(lo):
            pltpu.sync_copy(
                acc_s.at[pl.ds(sid * rows_per_tile, rows_per_tile)],
                out_h.at[cid].at[pl.ds(lo + sid * rows_per_tile,
                                       rows_per_tile)])

        def put_tgt(g, vals16):
            plsc.store_scatter(
                tgt_v, [g // 8 + 0 * _iota16(),
                        (g % 8) * LANES + _iota16()], vals16)

        def fire_scatters(vb):
            ds_ = [pltpu.async_copy(vb.at[pl.ds(k * 128, 128)],
                                    acc_s.at[tgt_v.at[k]], sems, add=True)
                   for k in range(4)]
            for d in ds_:
                d.wait()

        if n_ranges == 1:
            zero_slice()
            plsc.subcore_barrier()
            chunks = _static_chunks(n_contrib, SCR)
            vbufs = (vb0, vb1)
            sems_g = (semg0, semg1)
            dload = {}

            def start_load(ci):
                off, sz = chunks[ci]
                dload[ci] = pltpu.async_copy(
                    vals_h.at[pl.ds(tbase + off, sz)],
                    vbufs[ci % 2].at[pl.ds(0, sz)], sems_g[ci % 2])

            start_load(0)
            for ci, (off, sz) in enumerate(chunks):
                if ci + 1 < len(chunks):
                    start_load(ci + 1)
                dload.pop(ci).wait()
                ng = -(-sz // LANES)
                for g in range(ng):
                    iv = _vload(idx_v, off + g * LANES)
                    rem = sz - g * LANES
                    if rem < LANES:
                        iv = jnp.where(_iota16() < rem, iv, rng_size)
                    put_tgt(g, iv)
                for g in range(ng, SCR // LANES):
                    put_tgt(g, rng_size + 0 * _iota16())
                fire_scatters(vbufs[ci % 2])
            plsc.subcore_barrier()
            dump_slice(0)
        else:
            for r in range(n_ranges):
                zero_slice()
                lo = r * rng_size

                def scan_g(g, cnt):
                    iv = _vload(idx_v, g * LANES)
                    mask = lax.shift_right_logical(iv, shift) == r
                    plsc.store_compressed(
                        bkt_v.at[pl.ds(cnt, LANES)],
                        tbase + g * LANES + _iota16(), mask=mask)
                    return cnt + jnp.sum(mask.astype(i32))

                cnt = lax.fori_loop(0, n_contrib // LANES, scan_g, 0)
                for g in range(SCR // LANES):
                    _vstore(bkt_v, cnt + g * LANES, tbase + 0 * _iota16())
                plsc.subcore_barrier()

                def sc_loop(t, _):
                    soff = t * SCR
                    dg = [pltpu.async_copy(
                        vals_h.at[bkt_v.at[pl.ds(soff + k * 128, 128)]],
                        vb0.at[pl.ds(k * 128, 128)], semg0)
                        for k in range(4)]
                    for d in dg:
                        d.wait()

                    def tgt_g(g, _):
                        gpos = _vload(bkt_v, soff + g * LANES)
                        iv = plsc.load_gather(idx_v, [gpos - tbase])
                        valid = (soff + g * LANES + _iota16()) < cnt
                        put_tgt(g, jnp.where(valid, iv - lo, rng_size))
                        return 0

                    lax.fori_loop(0, SCR // LANES, tgt_g, 0)
                    fire_scatters(vb0)
                    return 0

                trips = lax.shift_right_logical(cnt + (SCR - 1), 9)
                lax.fori_loop(0, trips, sc_loop, 0)
                plsc.subcore_barrier()
                dump_slice(lo)
                plsc.subcore_barrier()

    kern = pl.kernel(
        body,
        out_type=jax.ShapeDtypeStruct((2, m_out, EMB), f32),
        mesh=plsc.VectorSubcoreMesh(core_axis_name="c", subcore_axis_name="s"),
        compiler_params=pltpu.CompilerParams(needs_layout_passes=False,
                                             use_tc_tiling_on_sc=False),
        scratch_types=[
            pltpu.VMEM((n_contrib + 16,), i32),
            pltpu.VMEM((n_contrib + SCR,), i32),
            pltpu.VMEM((4, 128), i32),
            pltpu.VMEM((SCR, EMB), f32),
            pltpu.VMEM((SCR, EMB), f32),
            pltpu.VMEM((128, EMB), f32),
            pltpu.VMEM_SHARED((rng_size + TRASH, EMB), f32),
            pltpu.SemaphoreType.DMA,
            pltpu.SemaphoreType.DMA,
            pltpu.SemaphoreType.DMA,
        ],
    )
    return kern


_segsum_edge_k = None
_segsum_atom_k = None


def _get_segsum_edge():
    global _segsum_edge_k
    if _segsum_edge_k is None:
        _segsum_edge_k = _make_segsum(NG, R_EDGE, NRANGE_EDGE, SHIFT_EDGE)
    return _segsum_edge_k


def _get_segsum_atom():
    global _segsum_atom_k
    if _segsum_atom_k is None:
        _segsum_atom_k = _make_segsum(NE, R_ATOM, 1, 0)
    return _segsum_atom_k


def _swish(x):
    return x / (1.0 + jnp.exp(-x))


def _envelope(x):
    p = 6
    a = -(p + 1) * (p + 2) / 2.0
    b = float(p * (p + 2))
    c = -p * (p + 1) / 2.0
    x2 = x * x
    x4 = x2 * x2
    x5 = x4 * x
    env = 1.0 / x + a * x5 + b * x5 * x + c * x5 * x2
    return jnp.where(x < 1.0, env, 0.0)


NE_PAD = 163840
NG_PAD = 327680


def _env_sin(dq):
    d = jnp.sqrt(dq + 1e-12)
    x = d / CUT
    env = _envelope(x)
    return [env * jnp.sin((n + 1.0) * jnp.pi * x) for n in range(6)]


def _tc_erbf_body(dq_ref, er_ref):
    rads = _env_sin(dq_ref[...])
    s = jnp.sqrt(2.0 / CUT)
    for n in range(6):
        er_ref[n] = rads[n] * s
    z = jnp.zeros_like(rads[0])
    er_ref[6] = z
    er_ref[7] = z


def _tc_erbf(dsq_pad):
    BB = 256
    return pl.pallas_call(
        _tc_erbf_body,
        grid=(NE_PAD // 128 // BB,),
        in_specs=[pl.BlockSpec((BB, 128), lambda i: (i, 0))],
        out_specs=pl.BlockSpec((8, BB, 128), lambda i: (0, i, 0)),
        out_shape=jax.ShapeDtypeStruct((8, NE_PAD // 128, 128), f32),
    )(dsq_pad)


def _tc_sbf_body(dq_ref, ad_ref, ac_ref, sbf_ref):
    rads = _env_sin(dq_ref[...])
    adot = ad_ref[...]
    acsq = ac_ref[...]
    cosa = adot * lax.rsqrt(adot * adot + acsq + 1e-12)
    ts = [jnp.ones_like(cosa), cosa]
    for _ in range(5):
        ts.append(2.0 * cosa * ts[-1] - ts[-2])
    for n in range(6):
        for l in range(7):
            sbf_ref[n * 7 + l] = (rads[n] * ts[l]).astype(bf16)
    z = jnp.zeros_like(rads[0]).astype(bf16)
    for p in range(42, 48):
        sbf_ref[p] = z


def _tc_sbf(dqk_pad, ad_pad, ac_pad):
    BB = 256
    return pl.pallas_call(
        _tc_sbf_body,
        grid=(NG_PAD // 128 // BB,),
        in_specs=[pl.BlockSpec((BB, 128), lambda i: (i, 0))] * 3,
        out_specs=pl.BlockSpec((48, BB, 128), lambda i: (0, i, 0)),
        out_shape=jax.ShapeDtypeStruct((48, NG_PAD // 128, 128), bf16),
    )(dqk_pad, ad_pad, ac_pad)


def _tc_edge_init_body(e1_ref, er_ref, werc_ref, b_ref, wr0_ref,
                       m_ref, g0_ref):
    er = er_ref[...]
    m = _swish(e1_ref[...] + _mxu(er, werc_ref[...]) + b_ref[...])
    m_ref[...] = m
    g0_ref[...] = m * _mxu(er, wr0_ref[...])


def _tc_edge_init(e1, er, werc8, bvec, wrbf0):
    BE = 2000
    return pl.pallas_call(
        _tc_edge_init_body,
        grid=(NE // BE,),
        in_specs=[
            pl.BlockSpec((BE, EMB), lambda i: (i, 0)),
            pl.BlockSpec((BE, 8), lambda i: (i, 0)),
            pl.BlockSpec((8, EMB), lambda i: (0, 0)),
            pl.BlockSpec((1, EMB), lambda i: (0, 0)),
            pl.BlockSpec((8, EMB), lambda i: (0, 0)),
        ],
        out_specs=[
            pl.BlockSpec((BE, EMB), lambda i: (i, 0)),
            pl.BlockSpec((BE, EMB), lambda i: (i, 0)),
        ],
        out_shape=[
            jax.ShapeDtypeStruct((NE, EMB), f32),
            jax.ShapeDtypeStruct((NE, EMB), f32),
        ],
    )(e1, er, werc8, bvec, wrbf0)


def _tc_layer_pre_body(m_ref, er_ref, w1_ref, b1_ref, w2_ref, b2_ref,
                       wr_ref, xji_ref, q_ref):
    m = m_ref[...]
    er = er_ref[...]
    xji_ref[...] = _swish(_mxu(m, w1_ref[...]) + b1_ref[...])
    q_ref[...] = (_swish(_mxu(m, w2_ref[...]) + b2_ref[...])
                  * _mxu(er, wr_ref[...])).astype(bf16)


def _tc_layer_pre(m, er, w1, b1, w2, b2, wrbf8):
    BE = 2000
    return pl.pallas_call(
        _tc_layer_pre_body,
        grid=(NE // BE,),
        in_specs=[
            pl.BlockSpec((BE, EMB), lambda i: (i, 0)),
            pl.BlockSpec((BE, 8), lambda i: (i, 0)),
            pl.BlockSpec((EMB, EMB), lambda i: (0, 0)),
            pl.BlockSpec((1, EMB), lambda i: (0, 0)),
            pl.BlockSpec((EMB, EMB), lambda i: (0, 0)),
            pl.BlockSpec((1, EMB), lambda i: (0, 0)),
            pl.BlockSpec((8, EMB), lambda i: (0, 0)),
        ],
        out_specs=[
            pl.BlockSpec((BE, EMB), lambda i: (i, 0)),
            pl.BlockSpec((BE, EMB), lambda i: (i, 0)),
        ],
        out_shape=[
            jax.ShapeDtypeStruct((NE, EMB), f32),
            jax.ShapeDtypeStruct((NE, EMB), bf16),
        ],
    )(m, er, w1, b1, w2, b2, wrbf8)


def _tc_bilinear_body(qk_ref, sbf_ref, wb_ref, wse_ref, s_ref, y_ref):
    h = _mxu(qk_ref[...], wb_ref[...])
    spb = _mxu(sbf_ref[...], wse_ref[...])
    y_ref[...] = _mxu(spb * h, s_ref[...])


def _tc_bilinear(qk, sbf48, wbcat, wspE, smat):
    BA = 4000
    return pl.pallas_call(
        _tc_bilinear_body,
        grid=(NG // BA,),
        in_specs=[
            pl.BlockSpec((BA, EMB), lambda i: (i, 0)),
            pl.BlockSpec((BA, 48), lambda i: (i, 0)),
            pl.BlockSpec((EMB, EMB * NBILIN), lambda i: (0, 0)),
            pl.BlockSpec((48, EMB * NBILIN), lambda i: (0, 0)),
            pl.BlockSpec((EMB * NBILIN, EMB), lambda i: (0, 0)),
        ],
        out_specs=pl.BlockSpec((BA, EMB), lambda i: (i, 0)),
        out_shape=jax.ShapeDtypeStruct((NG, EMB), f32),
    )(qk, sbf48, wbcat, wspE, smat)


def _tc_layer_post_body(m_ref, xji_ref, a0_ref, a1_ref, er_ref,
                        wres_ref, bres_ref, wr_ref, mn_ref, g_ref):
    u = xji_ref[...] + a0_ref[0] + a1_ref[0]
    mn = m_ref[...] + _swish(_mxu(u, wres_ref[...]) + bres_ref[...])
    mn_ref[...] = mn
    g_ref[...] = mn * _mxu(er_ref[...], wr_ref[...])


def _tc_layer_post(m, xji, aggp, er, wres, bres, wrbf8):
    BE = 2000
    return pl.pallas_call(
        _tc_layer_post_body,
        grid=(NE // BE,),
        in_specs=[
            pl.BlockSpec((BE, EMB), lambda i: (i, 0)),
            pl.BlockSpec((BE, EMB), lambda i: (i, 0)),
            pl.BlockSpec((1, BE, EMB), lambda i: (0, i, 0)),
            pl.BlockSpec((1, BE, EMB), lambda i: (1, i, 0)),
            pl.BlockSpec((BE, 8), lambda i: (i, 0)),
            pl.BlockSpec((EMB, EMB), lambda i: (0, 0)),
            pl.BlockSpec((1, EMB), lambda i: (0, 0)),
            pl.BlockSpec((8, EMB), lambda i: (0, 0)),
        ],
        out_specs=[
            pl.BlockSpec((BE, EMB), lambda i: (i, 0)),
            pl.BlockSpec((BE, EMB), lambda i: (i, 0)),
        ],
        out_shape=[
            jax.ShapeDtypeStruct((NE, EMB), f32),
            jax.ShapeDtypeStruct((NE, EMB), f32),
        ],
    )(m, xji, aggp, aggp, er, wres, bres, wrbf8)


def _tc_atom_body(*refs):
    i = pl.program_id(0)
    out_ref = refs[-1]
    BT = refs[0].shape[1]
    rows = i * BT + lax.broadcasted_iota(i32, (BT, 1), 0)
    valid = (rows < NA).astype(f32)
    total = jnp.zeros((), f32)
    for k in range(NCONV + 1):
        tp0 = refs[2 * k][0]
        tp1 = refs[2 * k + 1][0]
        wh = refs[8 + 3 * k][...]
        bh = refs[8 + 3 * k + 1][...]
        wo = refs[8 + 3 * k + 2][...]
        s = _swish(_mxu(tp0 + tp1, wh) + bh)
        contrib = _mxu(s, wo) * valid
        total = total + jnp.sum(contrib)

    @pl.when(i == 0)
    def _():
        out_ref[...] = jnp.zeros((1, 1), f32)

    out_ref[...] += jnp.reshape(total, (1, 1))


def _tc_atom(tps, wsets):
    BT = 2528
    in_specs = []
    args = []
    for tp in tps:
        in_specs += [pl.BlockSpec((1, BT, EMB), lambda i: (0, i, 0)),
                     pl.BlockSpec((1, BT, EMB), lambda i: (1, i, 0))]
        args += [tp, tp]
    for (wh, bh, wo) in wsets:
        in_specs += [pl.BlockSpec((EMB, EMB), lambda i: (0, 0)),
                     pl.BlockSpec((1, EMB), lambda i: (0, 0)),
                     pl.BlockSpec((EMB, 1), lambda i: (0, 0))]
        args += [wh, bh, wo]
    return pl.pallas_call(
        _tc_atom_body,
        grid=(R_ATOM // BT,),
        in_specs=in_specs,
        out_specs=pl.BlockSpec((1, 1), lambda i: (0, 0)),
        out_shape=jax.ShapeDtypeStruct((1, 1), f32),
    )(*args)


def kernel(nxyz, params, nbr_list, angle_list, num_atoms, ji_idx, kj_idx):
    xs = jnp.asarray(nxyz[:, 1], f32)
    ys = jnp.asarray(nxyz[:, 2], f32)
    zs = jnp.asarray(nxyz[:, 3], f32)
    za = nxyz[:, 0].astype(i32)
    nb0 = jnp.asarray(nbr_list[:, 0], i32)
    nb1 = jnp.asarray(nbr_list[:, 1], i32)
    g0 = jnp.asarray(angle_list[:, 0], i32)
    g1 = jnp.asarray(angle_list[:, 1], i32)
    g2 = jnp.asarray(angle_list[:, 2], i32)
    ji = jnp.asarray(ji_idx, i32)
    kj = jnp.asarray(kj_idx, i32)

    W = params["emb_W"]
    a1f = (params["emb"] @ W[0:EMB]).reshape(-1)
    a2f = (params["emb"] @ W[EMB:2 * EMB]).reshape(-1)
    werc8 = jnp.pad(params["emb_Wrbf"] @ W[2 * EMB:], ((0, 2), (0, 0)))
    bvec = params["emb_b"].reshape(1, EMB)

    def pad8(w):
        return jnp.pad(w, ((0, 2), (0, 0)))

    emat = jnp.kron(jnp.eye(NBILIN, dtype=f32), jnp.ones((1, EMB), f32))
    smat = jnp.kron(jnp.ones((NBILIN, 1), f32), jnp.eye(EMB, dtype=f32))
    wspE_list = []
    wb_list = []
    for l in range(NCONV):
        p = params["int"][l]
        wspE_list.append(jnp.pad(p["W_sbf"], ((0, 6), (0, 0))) @ emat)
        wb_list.append(jnp.transpose(p["W_bilin"], (1, 0, 2))
                       .reshape(EMB, NBILIN * EMB))

    dsq, e1f, adot, acsq = _sc_geom(xs, ys, zs, za, a1f, a2f,
                                    nb0, nb1, g0, g1, g2)
    e1 = e1f.reshape(NE, EMB)

    dsq_pad = jnp.pad(dsq, (0, NE_PAD - NE),
                      constant_values=1e6).reshape(NE_PAD // 128, 128)
    er = _tc_erbf(dsq_pad).reshape(8, NE_PAD).T
    m, g0e = _tc_edge_init(e1, er, werc8, bvec,
                           pad8(params["out"][0]["W_rbf"]))

    dqk = _sc_gather_scalar(dsq, kj)

    def _padg(a, v):
        return jnp.pad(a, (0, NG_PAD - NG),
                       constant_values=v).reshape(NG_PAD // 128, 128)

    sbf48 = _tc_sbf(_padg(dqk, 1e6), _padg(adot, 0.0),
                    _padg(acsq, 0.0)).reshape(48, NG_PAD).T

    segsum_atom = _get_segsum_atom()
    segsum_edge = _get_segsum_edge()
    tps = [segsum_atom(g0e, nb0)]

    for l in range(NCONV):
        p = params["int"][l]
        xji, q = _tc_layer_pre(m, er, p["W1"], p["b1"].reshape(1, EMB),
                               p["W2"], p["b2"].reshape(1, EMB),
                               pad8(p["W_rbf"]))
        qk = _sc_gather_rows(q, kj)
        y = _tc_bilinear(qk, sbf48, wb_list[l], wspE_list[l], smat)
        aggp = segsum_edge(y, ji)
        m, ge = _tc_layer_post(m, xji, aggp, er,
                               p["W_res"], p["b_res"].reshape(1, EMB),
                               pad8(params["out"][l + 1]["W_rbf"]))
        tps.append(segsum_atom(ge, nb0))

    wsets = [(po["W_h"], po["b_h"].reshape(1, EMB), po["W_out"])
             for po in params["out"]]
    total = _tc_atom(tps, wsets)
    return jnp.reshape(total, (1,))


# speedup vs baseline: 1.4197x; 1.0028x over previous
import functools

import jax
import jax.numpy as jnp
from jax import lax
from jax.experimental import pallas as pl
from jax.experimental.pallas import tpu as pltpu
from jax.experimental.pallas import tpu_sc as plsc

NA = 10000
NE = 160000
NG = 320000
EMB = 64
NRBF = 6
CUT = 5.0
NBILIN = 8
NCONV = 3

NW = 32
LANES = 16

f32 = jnp.float32
i32 = jnp.int32
bf16 = jnp.bfloat16


def _mxu(a, b):
    return jnp.dot(a.astype(bf16), b.astype(bf16), preferred_element_type=f32)

R_EDGE = 16384
SHIFT_EDGE = 14
NRANGE_EDGE = -(-NE // R_EDGE)
R_ATOM = 10112
TRASH = 16


def _iota16():
    return lax.iota(i32, LANES)


def _vload(ref, off):
    return plsc.load_gather(ref, [off + _iota16()])


def _vstore(ref, off, x, mask=None):
    plsc.store_scatter(ref, [off + _iota16()], x, mask=mask)


def _tile_chunk_range(wid, n_chunks):
    q, rem = divmod(n_chunks, NW)
    count = q + jnp.where(wid < rem, 1, 0)
    first = wid * q + jnp.minimum(wid, rem)
    return first, count


def _sc_geom_body(xs_h, ys_h, zs_h, za_h, a1_h, a2_h, nb0_h, nb1_h,
                  g0_h, g1_h, g2_h,
                  dsq_h, e1_h, adot_h, acsq_h,
                  xs_v, ys_v, zs_v, za_v, a1_v, a2_v,
                  eb0, eb1, dq_b, e1_b,
                  gb0, gb1, gb2, ad_b, ac_b):
    wid = lax.axis_index("s") * 2 + lax.axis_index("c")
    pltpu.sync_copy(xs_h, xs_v)
    pltpu.sync_copy(ys_h, ys_v)
    pltpu.sync_copy(zs_h, zs_v)
    pltpu.sync_copy(za_h, za_v)
    pltpu.sync_copy(a1_h, a1_v)
    pltpu.sync_copy(a2_h, a2_v)

    CE = 400
    first, count = _tile_chunk_range(wid, NE // CE)

    def edge_chunk(c, _):
        base = (first + c) * CE
        pltpu.sync_copy(nb0_h.at[pl.ds(base, CE)], eb0)
        pltpu.sync_copy(nb1_h.at[pl.ds(base, CE)], eb1)

        def grp(g, _):
            off = g * LANES
            s = _vload(eb0, off)
            t = _vload(eb1, off)
            dx = plsc.load_gather(xs_v, [s]) - plsc.load_gather(xs_v, [t])
            dy = plsc.load_gather(ys_v, [s]) - plsc.load_gather(ys_v, [t])
            dz = plsc.load_gather(zs_v, [s]) - plsc.load_gather(zs_v, [t])
            _vstore(dq_b, off, dx * dx + dy * dy + dz * dz)
            zi = plsc.load_gather(za_v, [s]) * EMB
            zj = plsc.load_gather(za_v, [t]) * EMB
            eoff = off * EMB + _iota16() * EMB
            for ccol in range(EMB):
                v = (plsc.load_gather(a1_v, [zi + ccol]) +
                     plsc.load_gather(a2_v, [zj + ccol]))
                plsc.store_scatter(e1_b, [eoff + ccol], v)
            return 0

        lax.fori_loop(0, CE // LANES, grp, 0)
        pltpu.sync_copy(dq_b, dsq_h.at[pl.ds(base, CE)])
        pltpu.sync_copy(e1_b, e1_h.at[pl.ds(base * EMB, CE * EMB)])
        return 0

    lax.fori_loop(0, count, edge_chunk, 0)

    CA = 512
    afirst, acount = _tile_chunk_range(wid, NG // CA)

    def ang_chunk(c, _):
        base = (afirst + c) * CA
        pltpu.sync_copy(g0_h.at[pl.ds(base, CA)], gb0)
        pltpu.sync_copy(g1_h.at[pl.ds(base, CA)], gb1)
        pltpu.sync_copy(g2_h.at[pl.ds(base, CA)], gb2)

        def grp(g, _):
            off = g * LANES
            ia = _vload(gb0, off)
            ib = _vload(gb1, off)
            ic = _vload(gb2, off)
            bx = plsc.load_gather(xs_v, [ib])
            by = plsc.load_gather(ys_v, [ib])
            bz = plsc.load_gather(zs_v, [ib])
            jx = plsc.load_gather(xs_v, [ia]) - bx
            jy = plsc.load_gather(ys_v, [ia]) - by
            jz = plsc.load_gather(zs_v, [ia]) - bz
            kx = plsc.load_gather(xs_v, [ic]) - bx
            ky = plsc.load_gather(ys_v, [ic]) - by
            kz = plsc.load_gather(zs_v, [ic]) - bz
            _vstore(ad_b, off, jx * kx + jy * ky + jz * kz)
            cx = jy * kz - jz * ky
            cy = jz * kx - jx * kz
            cz = jx * ky - jy * kx
            _vstore(ac_b, off, cx * cx + cy * cy + cz * cz)
            return 0

        lax.fori_loop(0, CA // LANES, grp, 0)
        pltpu.sync_copy(ad_b, adot_h.at[pl.ds(base, CA)])
        pltpu.sync_copy(ac_b, acsq_h.at[pl.ds(base, CA)])
        return 0

    lax.fori_loop(0, acount, ang_chunk, 0)


def _sc_geom(xs, ys, zs, za, a1f, a2f, nb0, nb1, g0, g1, g2):
    CE, CA = 400, 512
    kern = pl.kernel(
        _sc_geom_body,
        out_type=(
            jax.ShapeDtypeStruct((NE,), f32),
            jax.ShapeDtypeStruct((NE * EMB,), f32),
            jax.ShapeDtypeStruct((NG,), f32),
            jax.ShapeDtypeStruct((NG,), f32),
        ),
        mesh=plsc.VectorSubcoreMesh(core_axis_name="c", subcore_axis_name="s"),
        compiler_params=pltpu.CompilerParams(needs_layout_passes=False, use_tc_tiling_on_sc=False),
        scratch_types=[
            pltpu.VMEM((NA,), f32), pltpu.VMEM((NA,), f32),
            pltpu.VMEM((NA,), f32), pltpu.VMEM((NA,), i32),
            pltpu.VMEM((95 * EMB,), f32), pltpu.VMEM((95 * EMB,), f32),
            pltpu.VMEM((CE,), i32), pltpu.VMEM((CE,), i32),
            pltpu.VMEM((CE,), f32), pltpu.VMEM((CE * EMB,), f32),
            pltpu.VMEM((CA,), i32), pltpu.VMEM((CA,), i32),
            pltpu.VMEM((CA,), i32),
            pltpu.VMEM((CA,), f32), pltpu.VMEM((CA,), f32),
        ],
    )
    return kern(xs, ys, zs, za, a1f, a2f, nb0, nb1, g0, g1, g2)


def _make_gather(width):
    SCR = 512
    n_contrib = NG // NW

    def body(table_h, idx_h, out_h, idx_v, vb0, vb1, semg0, semg1):
        wid = lax.axis_index("s") * 2 + lax.axis_index("c")
        tbase = wid * n_contrib
        pltpu.sync_copy(idx_h.at[pl.ds(tbase, n_contrib)], idx_v)
        chunks = _static_chunks(n_contrib, SCR)
        vbufs = (vb0, vb1)
        sems_g = (semg0, semg1)
        dload = {}

        def start_load(ci):
            off, sz = chunks[ci]
            dload[ci] = [pltpu.async_copy(
                table_h.at[idx_v.at[pl.ds(off + koff, ksz)]],
                vbufs[ci % 2].at[pl.ds(koff, ksz)], sems_g[ci % 2])
                for koff, ksz in _static_chunks(sz, 128)]

        start_load(0)
        for ci, (off, sz) in enumerate(chunks):
            if ci + 1 < len(chunks):
                start_load(ci + 1)
            for d in dload.pop(ci):
                d.wait()
            pltpu.sync_copy(vbufs[ci % 2].at[pl.ds(0, sz)],
                            out_h.at[pl.ds(tbase + off, sz)])

    out_shape = (NG, EMB) if width else (NG,)
    buf_shape = (SCR, EMB) if width else (SCR,)
    dt = bf16 if width else f32
    kern = pl.kernel(
        body,
        out_type=jax.ShapeDtypeStruct(out_shape, dt),
        mesh=plsc.VectorSubcoreMesh(core_axis_name="c", subcore_axis_name="s"),
        compiler_params=pltpu.CompilerParams(needs_layout_passes=False,
                                             use_tc_tiling_on_sc=False),
        scratch_types=[
            pltpu.VMEM((n_contrib,), i32),
            pltpu.VMEM(buf_shape, dt),
            pltpu.VMEM(buf_shape, dt),
            pltpu.SemaphoreType.DMA,
            pltpu.SemaphoreType.DMA,
        ],
    )
    return kern


_gather_rows_k = None
_gather_scalar_k = None


def _sc_gather_rows(table, idx):
    global _gather_rows_k
    if _gather_rows_k is None:
        _gather_rows_k = _make_gather(EMB)
    return _gather_rows_k(table, idx)


def _sc_gather_scalar(table, idx):
    global _gather_scalar_k
    if _gather_scalar_k is None:
        _gather_scalar_k = _make_gather(0)
    return _gather_scalar_k(table, idx)


def _static_chunks(total, c):
    out_list = []
    off = 0
    while off < total:
        sz = min(c, total - off)
        out_list.append((off, sz))
        off += sz
    return out_list


def _make_segsum(n_in, rng_size, n_ranges, shift):
    SCR = 512
    n_contrib = n_in // NW
    rows_per_tile = rng_size // LANES
    m_out = rng_size * n_ranges

    def body(vals_h, idx_h, out_h, idx_v, bkt_v, tgt_v, vb0, vb1, zbuf_v,
             acc_s, semg0, semg1, sems):
        cid = lax.axis_index("c")
        sid = lax.axis_index("s")
        wid = sid * 2 + cid
        tbase = wid * n_contrib
        pltpu.sync_copy(idx_h.at[pl.ds(tbase, n_contrib)],
                        idx_v.at[pl.ds(0, n_contrib)])

        def zinit(i, _):
            zbuf_v[i // 4, pl.ds((i % 4) * LANES, LANES)] = (
                jnp.zeros((LANES,), f32))
            return 0

        lax.fori_loop(0, 128 * 4, zinit, 0)

        def zero_slice():
            for (zoff, zsz) in _static_chunks(rows_per_tile, 128):
                pltpu.sync_copy(
                    zbuf_v.at[pl.ds(0, zsz)],
                    acc_s.at[pl.ds(sid * rows_per_tile + zoff, zsz)])

        def dump_slice(lo):
            pltpu.sync_copy(
                acc_s.at[pl.ds(sid * rows_per_tile, rows_per_tile)],
                out_h.at[cid].at[pl.ds(lo + sid * rows_per_tile,
                                       rows_per_tile)])

        def put_tgt(g, vals16):
            plsc.store_scatter(
                tgt_v, [g // 8 + 0 * _iota16(),
                        (g % 8) * LANES + _iota16()], vals16)

        def fire_scatters(vb):
            ds_ = [pltpu.async_copy(vb.at[pl.ds(k * 128, 128)],
                                    acc_s.at[tgt_v.at[k]], sems, add=True)
                   for k in range(4)]
            for d in ds_:
                d.wait()

        if n_ranges == 1:
            zero_slice()
            plsc.subcore_barrier()
            chunks = _static_chunks(n_contrib, SCR)
            vbufs = (vb0, vb1)
            sems_g = (semg0, semg1)
            dload = {}

            def start_load(ci):
                off, sz = chunks[ci]
                dload[ci] = pltpu.async_copy(
                    vals_h.at[pl.ds(tbase + off, sz)],
                    vbufs[ci % 2].at[pl.ds(0, sz)], sems_g[ci % 2])

            start_load(0)
            for ci, (off, sz) in enumerate(chunks):
                if ci + 1 < len(chunks):
                    start_load(ci + 1)
                dload.pop(ci).wait()
                ng = -(-sz // LANES)
                for g in range(ng):
                    iv = _vload(idx_v, off + g * LANES)
                    rem = sz - g * LANES
                    if rem < LANES:
                        iv = jnp.where(_iota16() < rem, iv, rng_size)
                    put_tgt(g, iv)
                for g in range(ng, SCR // LANES):
                    put_tgt(g, rng_size + 0 * _iota16())
                fire_scatters(vbufs[ci % 2])
            plsc.subcore_barrier()
            dump_slice(0)
        else:
            for r in range(n_ranges):
                zero_slice()
                lo = r * rng_size

                def scan_g(g, cnt):
                    iv = _vload(idx_v, g * LANES)
                    mask = lax.shift_right_logical(iv, shift) == r
                    plsc.store_compressed(
                        bkt_v.at[pl.ds(cnt, LANES)],
                        tbase + g * LANES + _iota16(), mask=mask)
                    return cnt + jnp.sum(mask.astype(i32))

                cnt = lax.fori_loop(0, n_contrib // LANES, scan_g, 0)
                for g in range(SCR // LANES):
                    _vstore(bkt_v, cnt + g * LANES, tbase + 0 * _iota16())
                plsc.subcore_barrier()

                def sc_loop(t, _):
                    soff = t * SCR
                    dg = [pltpu.async_copy(
                        vals_h.at[bkt_v.at[pl.ds(soff + k * 128, 128)]],
                        vb0.at[pl.ds(k * 128, 128)], semg0)
                        for k in range(4)]
                    for d in dg:
                        d.wait()

                    def tgt_g(g, _):
                        gpos = _vload(bkt_v, soff + g * LANES)
                        iv = plsc.load_gather(idx_v, [gpos - tbase])
                        valid = (soff + g * LANES + _iota16()) < cnt
                        put_tgt(g, jnp.where(valid, iv - lo, rng_size))
                        return 0

                    lax.fori_loop(0, SCR // LANES, tgt_g, 0)
                    fire_scatters(vb0)
                    return 0

                trips = lax.shift_right_logical(cnt + (SCR - 1), 9)
                lax.fori_loop(0, trips, sc_loop, 0)
                plsc.subcore_barrier()
                dump_slice(lo)
                plsc.subcore_barrier()

    kern = pl.kernel(
        body,
        out_type=jax.ShapeDtypeStruct((2, m_out, EMB), f32),
        mesh=plsc.VectorSubcoreMesh(core_axis_name="c", subcore_axis_name="s"),
        compiler_params=pltpu.CompilerParams(needs_layout_passes=False,
                                             use_tc_tiling_on_sc=False),
        scratch_types=[
            pltpu.VMEM((n_contrib + 16,), i32),
            pltpu.VMEM((n_contrib + SCR,), i32),
            pltpu.VMEM((4, 128), i32),
            pltpu.VMEM((SCR, EMB), f32),
            pltpu.VMEM((SCR, EMB), f32),
            pltpu.VMEM((128, EMB), f32),
            pltpu.VMEM_SHARED((rng_size + TRASH, EMB), f32),
            pltpu.SemaphoreType.DMA,
            pltpu.SemaphoreType.DMA,
            pltpu.SemaphoreType.DMA,
        ],
    )
    return kern


_segsum_edge_k = None
_segsum_atom_k = None


def _get_segsum_edge():
    global _segsum_edge_k
    if _segsum_edge_k is None:
        _segsum_edge_k = _make_segsum(NG, R_EDGE, NRANGE_EDGE, SHIFT_EDGE)
    return _segsum_edge_k


def _get_segsum_atom():
    global _segsum_atom_k
    if _segsum_atom_k is None:
        _segsum_atom_k = _make_segsum(NE, R_ATOM, 1, 0)
    return _segsum_atom_k


def _swish(x):
    return x / (1.0 + jnp.exp(-x))


def _envelope(x):
    p = 6
    a = -(p + 1) * (p + 2) / 2.0
    b = float(p * (p + 2))
    c = -p * (p + 1) / 2.0
    x2 = x * x
    x4 = x2 * x2
    x5 = x4 * x
    env = 1.0 / x + a * x5 + b * x5 * x + c * x5 * x2
    return jnp.where(x < 1.0, env, 0.0)


NE_PAD = 163840
NG_PAD = 327680


def _env_sin(dq):
    d = jnp.sqrt(dq + 1e-12)
    x = d / CUT
    env = _envelope(x)
    return [env * jnp.sin((n + 1.0) * jnp.pi * x) for n in range(6)]


def _tc_erbf_body(dq_ref, er_ref):
    rads = _env_sin(dq_ref[...])
    s = jnp.sqrt(2.0 / CUT)
    for n in range(6):
        er_ref[n] = rads[n] * s
    z = jnp.zeros_like(rads[0])
    er_ref[6] = z
    er_ref[7] = z


def _tc_erbf(dsq_pad):
    BB = 256
    return pl.pallas_call(
        _tc_erbf_body,
        grid=(NE_PAD // 128 // BB,),
        in_specs=[pl.BlockSpec((BB, 128), lambda i: (i, 0))],
        out_specs=pl.BlockSpec((8, BB, 128), lambda i: (0, i, 0)),
        out_shape=jax.ShapeDtypeStruct((8, NE_PAD // 128, 128), f32),
    )(dsq_pad)


def _tc_sbf_body(dq_ref, ad_ref, ac_ref, sbf_ref):
    rads = _env_sin(dq_ref[...])
    adot = ad_ref[...]
    acsq = ac_ref[...]
    cosa = adot * lax.rsqrt(adot * adot + acsq + 1e-12)
    ts = [jnp.ones_like(cosa), cosa]
    for _ in range(5):
        ts.append(2.0 * cosa * ts[-1] - ts[-2])
    for n in range(6):
        for l in range(7):
            sbf_ref[n * 7 + l] = (rads[n] * ts[l]).astype(bf16)
    z = jnp.zeros_like(rads[0]).astype(bf16)
    for p in range(42, 48):
        sbf_ref[p] = z


def _tc_sbf(dqk_pad, ad_pad, ac_pad):
    BB = 256
    return pl.pallas_call(
        _tc_sbf_body,
        grid=(NG_PAD // 128 // BB,),
        in_specs=[pl.BlockSpec((BB, 128), lambda i: (i, 0))] * 3,
        out_specs=pl.BlockSpec((48, BB, 128), lambda i: (0, i, 0)),
        out_shape=jax.ShapeDtypeStruct((48, NG_PAD // 128, 128), bf16),
    )(dqk_pad, ad_pad, ac_pad)


def _tc_edge_init_body(e1_ref, er_ref, werc_ref, b_ref, wr0_ref,
                       m_ref, g0_ref):
    er = er_ref[...]
    m = _swish(e1_ref[...] + _mxu(er, werc_ref[...]) + b_ref[...])
    m_ref[...] = m
    g0_ref[...] = m * _mxu(er, wr0_ref[...])


def _tc_edge_init(e1, er, werc8, bvec, wrbf0):
    BE = 2000
    return pl.pallas_call(
        _tc_edge_init_body,
        grid=(NE // BE,),
        in_specs=[
            pl.BlockSpec((BE, EMB), lambda i: (i, 0)),
            pl.BlockSpec((BE, 8), lambda i: (i, 0)),
            pl.BlockSpec((8, EMB), lambda i: (0, 0)),
            pl.BlockSpec((1, EMB), lambda i: (0, 0)),
            pl.BlockSpec((8, EMB), lambda i: (0, 0)),
        ],
        out_specs=[
            pl.BlockSpec((BE, EMB), lambda i: (i, 0)),
            pl.BlockSpec((BE, EMB), lambda i: (i, 0)),
        ],
        out_shape=[
            jax.ShapeDtypeStruct((NE, EMB), f32),
            jax.ShapeDtypeStruct((NE, EMB), f32),
        ],
    )(e1, er, werc8, bvec, wrbf0)


def _tc_layer_pre_body(m_ref, er_ref, w1_ref, b1_ref, w2_ref, b2_ref,
                       wr_ref, xji_ref, q_ref):
    m = m_ref[...]
    er = er_ref[...]
    xji_ref[...] = _swish(_mxu(m, w1_ref[...]) + b1_ref[...])
    q_ref[...] = (_swish(_mxu(m, w2_ref[...]) + b2_ref[...])
                  * _mxu(er, wr_ref[...])).astype(bf16)


def _tc_layer_pre(m, er, w1, b1, w2, b2, wrbf8):
    BE = 2000
    return pl.pallas_call(
        _tc_layer_pre_body,
        grid=(NE // BE,),
        in_specs=[
            pl.BlockSpec((BE, EMB), lambda i: (i, 0)),
            pl.BlockSpec((BE, 8), lambda i: (i, 0)),
            pl.BlockSpec((EMB, EMB), lambda i: (0, 0)),
            pl.BlockSpec((1, EMB), lambda i: (0, 0)),
            pl.BlockSpec((EMB, EMB), lambda i: (0, 0)),
            pl.BlockSpec((1, EMB), lambda i: (0, 0)),
            pl.BlockSpec((8, EMB), lambda i: (0, 0)),
        ],
        out_specs=[
            pl.BlockSpec((BE, EMB), lambda i: (i, 0)),
            pl.BlockSpec((BE, EMB), lambda i: (i, 0)),
        ],
        out_shape=[
            jax.ShapeDtypeStruct((NE, EMB), f32),
            jax.ShapeDtypeStruct((NE, EMB), bf16),
        ],
    )(m, er, w1, b1, w2, b2, wrbf8)


def _tc_bilinear_body(qk_ref, sbf_ref, wb_ref, wse_ref, y_ref):
    h = _mxu(qk_ref[...], wb_ref[...])
    spb = _mxu(sbf_ref[...], wse_ref[...])
    u = spb * h
    y = u[:, 0:EMB]
    for b in range(1, NBILIN):
        y = y + u[:, b * EMB:(b + 1) * EMB]
    y_ref[...] = y


def _tc_bilinear(qk, sbf48, wbcat, wspE):
    BA = 4000
    return pl.pallas_call(
        _tc_bilinear_body,
        grid=(NG // BA,),
        in_specs=[
            pl.BlockSpec((BA, EMB), lambda i: (i, 0)),
            pl.BlockSpec((BA, 48), lambda i: (i, 0)),
            pl.BlockSpec((EMB, EMB * NBILIN), lambda i: (0, 0)),
            pl.BlockSpec((48, EMB * NBILIN), lambda i: (0, 0)),
        ],
        out_specs=pl.BlockSpec((BA, EMB), lambda i: (i, 0)),
        out_shape=jax.ShapeDtypeStruct((NG, EMB), f32),
    )(qk, sbf48, wbcat, wspE)


def _tc_layer_post_body(m_ref, xji_ref, a0_ref, a1_ref, er_ref,
                        wres_ref, bres_ref, wr_ref, mn_ref, g_ref):
    u = xji_ref[...] + a0_ref[0] + a1_ref[0]
    mn = m_ref[...] + _swish(_mxu(u, wres_ref[...]) + bres_ref[...])
    mn_ref[...] = mn
    g_ref[...] = mn * _mxu(er_ref[...], wr_ref[...])


def _tc_layer_post(m, xji, aggp, er, wres, bres, wrbf8):
    BE = 2000
    return pl.pallas_call(
        _tc_layer_post_body,
        grid=(NE // BE,),
        in_specs=[
            pl.BlockSpec((BE, EMB), lambda i: (i, 0)),
            pl.BlockSpec((BE, EMB), lambda i: (i, 0)),
            pl.BlockSpec((1, BE, EMB), lambda i: (0, i, 0)),
            pl.BlockSpec((1, BE, EMB), lambda i: (1, i, 0)),
            pl.BlockSpec((BE, 8), lambda i: (i, 0)),
            pl.BlockSpec((EMB, EMB), lambda i: (0, 0)),
            pl.BlockSpec((1, EMB), lambda i: (0, 0)),
            pl.BlockSpec((8, EMB), lambda i: (0, 0)),
        ],
        out_specs=[
            pl.BlockSpec((BE, EMB), lambda i: (i, 0)),
            pl.BlockSpec((BE, EMB), lambda i: (i, 0)),
        ],
        out_shape=[
            jax.ShapeDtypeStruct((NE, EMB), f32),
            jax.ShapeDtypeStruct((NE, EMB), f32),
        ],
    )(m, xji, aggp, aggp, er, wres, bres, wrbf8)


def _tc_atom_body(*refs):
    i = pl.program_id(0)
    out_ref = refs[-1]
    BT = refs[0].shape[1]
    rows = i * BT + lax.broadcasted_iota(i32, (BT, 1), 0)
    valid = (rows < NA).astype(f32)
    total = jnp.zeros((), f32)
    for k in range(NCONV + 1):
        tp0 = refs[2 * k][0]
        tp1 = refs[2 * k + 1][0]
        wh = refs[8 + 3 * k][...]
        bh = refs[8 + 3 * k + 1][...]
        wo = refs[8 + 3 * k + 2][...]
        s = _swish(_mxu(tp0 + tp1, wh) + bh)
        contrib = _mxu(s, wo) * valid
        total = total + jnp.sum(contrib)

    @pl.when(i == 0)
    def _():
        out_ref[...] = jnp.zeros((1, 1), f32)

    out_ref[...] += jnp.reshape(total, (1, 1))


def _tc_atom(tps, wsets):
    BT = 2528
    in_specs = []
    args = []
    for tp in tps:
        in_specs += [pl.BlockSpec((1, BT, EMB), lambda i: (0, i, 0)),
                     pl.BlockSpec((1, BT, EMB), lambda i: (1, i, 0))]
        args += [tp, tp]
    for (wh, bh, wo) in wsets:
        in_specs += [pl.BlockSpec((EMB, EMB), lambda i: (0, 0)),
                     pl.BlockSpec((1, EMB), lambda i: (0, 0)),
                     pl.BlockSpec((EMB, 1), lambda i: (0, 0))]
        args += [wh, bh, wo]
    return pl.pallas_call(
        _tc_atom_body,
        grid=(R_ATOM // BT,),
        in_specs=in_specs,
        out_specs=pl.BlockSpec((1, 1), lambda i: (0, 0)),
        out_shape=jax.ShapeDtypeStruct((1, 1), f32),
    )(*args)


def kernel(nxyz, params, nbr_list, angle_list, num_atoms, ji_idx, kj_idx):
    xs = jnp.asarray(nxyz[:, 1], f32)
    ys = jnp.asarray(nxyz[:, 2], f32)
    zs = jnp.asarray(nxyz[:, 3], f32)
    za = nxyz[:, 0].astype(i32)
    nb0 = jnp.asarray(nbr_list[:, 0], i32)
    nb1 = jnp.asarray(nbr_list[:, 1], i32)
    g0 = jnp.asarray(angle_list[:, 0], i32)
    g1 = jnp.asarray(angle_list[:, 1], i32)
    g2 = jnp.asarray(angle_list[:, 2], i32)
    ji = jnp.asarray(ji_idx, i32)
    kj = jnp.asarray(kj_idx, i32)

    W = params["emb_W"]
    a1f = (params["emb"] @ W[0:EMB]).reshape(-1)
    a2f = (params["emb"] @ W[EMB:2 * EMB]).reshape(-1)
    werc8 = jnp.pad(params["emb_Wrbf"] @ W[2 * EMB:], ((0, 2), (0, 0)))
    bvec = params["emb_b"].reshape(1, EMB)

    def pad8(w):
        return jnp.pad(w, ((0, 2), (0, 0)))

    emat = jnp.kron(jnp.eye(NBILIN, dtype=f32), jnp.ones((1, EMB), f32))
    wspE_list = []
    wb_list = []
    for l in range(NCONV):
        p = params["int"][l]
        wspE_list.append(jnp.pad(p["W_sbf"], ((0, 6), (0, 0))) @ emat)
        wb_list.append(jnp.transpose(p["W_bilin"], (1, 0, 2))
                       .reshape(EMB, NBILIN * EMB))

    dsq, e1f, adot, acsq = _sc_geom(xs, ys, zs, za, a1f, a2f,
                                    nb0, nb1, g0, g1, g2)
    e1 = e1f.reshape(NE, EMB)

    dsq_pad = jnp.pad(dsq, (0, NE_PAD - NE),
                      constant_values=1e6).reshape(NE_PAD // 128, 128)
    er = _tc_erbf(dsq_pad).reshape(8, NE_PAD).T
    m, g0e = _tc_edge_init(e1, er, werc8, bvec,
                           pad8(params["out"][0]["W_rbf"]))

    dqk = _sc_gather_scalar(dsq, kj)

    def _padg(a, v):
        return jnp.pad(a, (0, NG_PAD - NG),
                       constant_values=v).reshape(NG_PAD // 128, 128)

    sbf48 = _tc_sbf(_padg(dqk, 1e6), _padg(adot, 0.0),
                    _padg(acsq, 0.0)).reshape(48, NG_PAD).T

    segsum_atom = _get_segsum_atom()
    segsum_edge = _get_segsum_edge()
    tps = [segsum_atom(g0e, nb0)]

    for l in range(NCONV):
        p = params["int"][l]
        xji, q = _tc_layer_pre(m, er, p["W1"], p["b1"].reshape(1, EMB),
                               p["W2"], p["b2"].reshape(1, EMB),
                               pad8(p["W_rbf"]))
        qk = _sc_gather_rows(q, kj)
        y = _tc_bilinear(qk, sbf48, wb_list[l], wspE_list[l])
        aggp = segsum_edge(y, ji)
        m, ge = _tc_layer_post(m, xji, aggp, er,
                               p["W_res"], p["b_res"].reshape(1, EMB),
                               pad8(params["out"][l + 1]["W_rbf"]))
        tps.append(segsum_atom(ge, nb0))

    wsets = [(po["W_h"], po["b_h"].reshape(1, EMB), po["W_out"])
             for po in params["out"]]
    total = _tc_atom(tps, wsets)
    return jnp.reshape(total, (1,))


# speedup vs baseline: 1.5475x; 1.0900x over previous
import functools

import jax
import jax.numpy as jnp
from jax import lax
from jax.experimental import pallas as pl
from jax.experimental.pallas import tpu as pltpu
from jax.experimental.pallas import tpu_sc as plsc

NA = 10000
NE = 160000
NG = 320000
EMB = 64
NRBF = 6
CUT = 5.0
NBILIN = 8
NCONV = 3

NW = 32
LANES = 16

f32 = jnp.float32
i32 = jnp.int32
bf16 = jnp.bfloat16


def _mxu(a, b):
    return jnp.dot(a.astype(bf16), b.astype(bf16), preferred_element_type=f32)

R_EDGE = 16384
SHIFT_EDGE = 14
NRANGE_EDGE = -(-NE // R_EDGE)
R_ATOM = 10112
TRASH = 16


def _iota16():
    return lax.iota(i32, LANES)


def _vload(ref, off):
    return plsc.load_gather(ref, [off + _iota16()])


def _vstore(ref, off, x, mask=None):
    plsc.store_scatter(ref, [off + _iota16()], x, mask=mask)


def _tile_chunk_range(wid, n_chunks):
    q, rem = divmod(n_chunks, NW)
    count = q + jnp.where(wid < rem, 1, 0)
    first = wid * q + jnp.minimum(wid, rem)
    return first, count


NA_P = 10016


def _sc_geom_body(xs_h, ys_h, zs_h, za_h, a1_h, a2_h, nb0_h, nb1_h,
                  g0_h, g1_h, g2_h,
                  dsq_h, adot_h, acsq_h, p1_h, p2_h,
                  xs_v, ys_v, zs_v, za_v, a1_v, a2_v,
                  eb0, eb1, dq_b, pb1, pb2,
                  gb0, gb1, gb2, ad_b, ac_b):
    wid = lax.axis_index("s") * 2 + lax.axis_index("c")
    pltpu.sync_copy(xs_h, xs_v)
    pltpu.sync_copy(ys_h, ys_v)
    pltpu.sync_copy(zs_h, zs_v)
    pltpu.sync_copy(za_h, za_v.at[pl.ds(0, NA)])
    pltpu.sync_copy(a1_h, a1_v)
    pltpu.sync_copy(a2_h, a2_v)

    APT = NA_P // NW
    abase = wid * APT

    def atom_grp(g, _):
        zraw = plsc.load_gather(za_v, [abase + g * LANES + _iota16()])
        zc = jnp.minimum(jnp.maximum(zraw, 0), 94) * EMB
        roff = g * LANES * EMB + _iota16() * EMB
        for ccol in range(EMB):
            plsc.store_scatter(pb1, [roff + ccol],
                               plsc.load_gather(a1_v, [zc + ccol]))
            plsc.store_scatter(pb2, [roff + ccol],
                               plsc.load_gather(a2_v, [zc + ccol]))
        return 0

    lax.fori_loop(0, APT // LANES + 1, atom_grp, 0)
    pltpu.sync_copy(pb1.at[pl.ds(0, APT * EMB)],
                    p1_h.at[pl.ds(abase * EMB, APT * EMB)])
    pltpu.sync_copy(pb2.at[pl.ds(0, APT * EMB)],
                    p2_h.at[pl.ds(abase * EMB, APT * EMB)])

    CE = 400
    first, count = _tile_chunk_range(wid, NE // CE)

    def edge_chunk(c, _):
        base = (first + c) * CE
        pltpu.sync_copy(nb0_h.at[pl.ds(base, CE)], eb0)
        pltpu.sync_copy(nb1_h.at[pl.ds(base, CE)], eb1)

        def grp(g, _):
            off = g * LANES
            s = _vload(eb0, off)
            t = _vload(eb1, off)
            dx = plsc.load_gather(xs_v, [s]) - plsc.load_gather(xs_v, [t])
            dy = plsc.load_gather(ys_v, [s]) - plsc.load_gather(ys_v, [t])
            dz = plsc.load_gather(zs_v, [s]) - plsc.load_gather(zs_v, [t])
            _vstore(dq_b, off, dx * dx + dy * dy + dz * dz)
            return 0

        lax.fori_loop(0, CE // LANES, grp, 0)
        pltpu.sync_copy(dq_b, dsq_h.at[pl.ds(base, CE)])
        return 0

    lax.fori_loop(0, count, edge_chunk, 0)

    CA = 512
    afirst, acount = _tile_chunk_range(wid, NG // CA)

    def ang_chunk(c, _):
        base = (afirst + c) * CA
        pltpu.sync_copy(g0_h.at[pl.ds(base, CA)], gb0)
        pltpu.sync_copy(g1_h.at[pl.ds(base, CA)], gb1)
        pltpu.sync_copy(g2_h.at[pl.ds(base, CA)], gb2)

        def grp(g, _):
            off = g * LANES
            ia = _vload(gb0, off)
            ib = _vload(gb1, off)
            ic = _vload(gb2, off)
            bx = plsc.load_gather(xs_v, [ib])
            by = plsc.load_gather(ys_v, [ib])
            bz = plsc.load_gather(zs_v, [ib])
            jx = plsc.load_gather(xs_v, [ia]) - bx
            jy = plsc.load_gather(ys_v, [ia]) - by
            jz = plsc.load_gather(zs_v, [ia]) - bz
            kx = plsc.load_gather(xs_v, [ic]) - bx
            ky = plsc.load_gather(ys_v, [ic]) - by
            kz = plsc.load_gather(zs_v, [ic]) - bz
            _vstore(ad_b, off, jx * kx + jy * ky + jz * kz)
            cx = jy * kz - jz * ky
            cy = jz * kx - jx * kz
            cz = jx * ky - jy * kx
            _vstore(ac_b, off, cx * cx + cy * cy + cz * cz)
            return 0

        lax.fori_loop(0, CA // LANES, grp, 0)
        pltpu.sync_copy(ad_b, adot_h.at[pl.ds(base, CA)])
        pltpu.sync_copy(ac_b, acsq_h.at[pl.ds(base, CA)])
        return 0

    lax.fori_loop(0, acount, ang_chunk, 0)


def _sc_geom(xs, ys, zs, za, a1f, a2f, nb0, nb1, g0, g1, g2):
    CE, CA = 400, 512
    APT = NA_P // NW
    kern = pl.kernel(
        _sc_geom_body,
        out_type=(
            jax.ShapeDtypeStruct((NE,), f32),
            jax.ShapeDtypeStruct((NG,), f32),
            jax.ShapeDtypeStruct((NG,), f32),
            jax.ShapeDtypeStruct((NA_P * EMB,), f32),
            jax.ShapeDtypeStruct((NA_P * EMB,), f32),
        ),
        mesh=plsc.VectorSubcoreMesh(core_axis_name="c", subcore_axis_name="s"),
        compiler_params=pltpu.CompilerParams(needs_layout_passes=False,
                                             use_tc_tiling_on_sc=False),
        scratch_types=[
            pltpu.VMEM((NA,), f32), pltpu.VMEM((NA,), f32),
            pltpu.VMEM((NA,), f32), pltpu.VMEM((NA_P + LANES,), i32),
            pltpu.VMEM((95 * EMB,), f32), pltpu.VMEM((95 * EMB,), f32),
            pltpu.VMEM((CE,), i32), pltpu.VMEM((CE,), i32),
            pltpu.VMEM((CE,), f32),
            pltpu.VMEM(((APT + LANES) * EMB,), f32),
            pltpu.VMEM(((APT + LANES) * EMB,), f32),
            pltpu.VMEM((CA,), i32), pltpu.VMEM((CA,), i32),
            pltpu.VMEM((CA,), i32),
            pltpu.VMEM((CA,), f32), pltpu.VMEM((CA,), f32),
        ],
    )
    return kern(xs, ys, zs, za, a1f, a2f, nb0, nb1, g0, g1, g2)


def _make_gather(width, n_out=NG, dtype_f32=False):
    SCR = 512
    n_contrib = n_out // NW

    def body(table_h, idx_h, out_h, idx_v, vb0, vb1, semg0, semg1):
        wid = lax.axis_index("s") * 2 + lax.axis_index("c")
        tbase = wid * n_contrib
        pltpu.sync_copy(idx_h.at[pl.ds(tbase, n_contrib)], idx_v)
        chunks = _static_chunks(n_contrib, SCR)
        vbufs = (vb0, vb1)
        sems_g = (semg0, semg1)
        dload = {}

        def start_load(ci):
            off, sz = chunks[ci]
            dload[ci] = [pltpu.async_copy(
                table_h.at[idx_v.at[pl.ds(off + koff, ksz)]],
                vbufs[ci % 2].at[pl.ds(koff, ksz)], sems_g[ci % 2])
                for koff, ksz in _static_chunks(sz, 128)]

        start_load(0)
        for ci, (off, sz) in enumerate(chunks):
            if ci + 1 < len(chunks):
                start_load(ci + 1)
            for d in dload.pop(ci):
                d.wait()
            pltpu.sync_copy(vbufs[ci % 2].at[pl.ds(0, sz)],
                            out_h.at[pl.ds(tbase + off, sz)])

    out_shape = (n_out, EMB) if width else (n_out,)
    buf_shape = (SCR, EMB) if width else (SCR,)
    dt = f32 if (dtype_f32 or not width) else bf16
    kern = pl.kernel(
        body,
        out_type=jax.ShapeDtypeStruct(out_shape, dt),
        mesh=plsc.VectorSubcoreMesh(core_axis_name="c", subcore_axis_name="s"),
        compiler_params=pltpu.CompilerParams(needs_layout_passes=False,
                                             use_tc_tiling_on_sc=False),
        scratch_types=[
            pltpu.VMEM((n_contrib,), i32),
            pltpu.VMEM(buf_shape, dt),
            pltpu.VMEM(buf_shape, dt),
            pltpu.SemaphoreType.DMA,
            pltpu.SemaphoreType.DMA,
        ],
    )
    return kern


_gather_cache = {}


def _get_gather(width, n_out=NG, dtype_f32=False):
    key = (width, n_out, dtype_f32)
    if key not in _gather_cache:
        _gather_cache[key] = _make_gather(width, n_out, dtype_f32)
    return _gather_cache[key]


def _sc_gather_rows(table, idx):
    return _get_gather(EMB)(table, idx)


def _sc_gather_scalar(table, idx):
    return _get_gather(0)(table, idx)


def _static_chunks(total, c):
    out_list = []
    off = 0
    while off < total:
        sz = min(c, total - off)
        out_list.append((off, sz))
        off += sz
    return out_list


def _make_segsum(n_in, rng_size, n_ranges, shift):
    SCR = 512
    n_contrib = n_in // NW
    rows_per_tile = rng_size // LANES
    m_out = rng_size * n_ranges

    def body(vals_h, idx_h, out_h, idx_v, bkt_v, tgt_v, vb0, vb1, zbuf_v,
             acc_s, semg0, semg1, sems):
        cid = lax.axis_index("c")
        sid = lax.axis_index("s")
        wid = sid * 2 + cid
        tbase = wid * n_contrib
        pltpu.sync_copy(idx_h.at[pl.ds(tbase, n_contrib)],
                        idx_v.at[pl.ds(0, n_contrib)])

        def zinit(i, _):
            zbuf_v[i // 4, pl.ds((i % 4) * LANES, LANES)] = (
                jnp.zeros((LANES,), f32))
            return 0

        lax.fori_loop(0, 128 * 4, zinit, 0)

        def zero_slice():
            for (zoff, zsz) in _static_chunks(rows_per_tile, 128):
                pltpu.sync_copy(
                    zbuf_v.at[pl.ds(0, zsz)],
                    acc_s.at[pl.ds(sid * rows_per_tile + zoff, zsz)])

        def dump_slice(lo):
            pltpu.sync_copy(
                acc_s.at[pl.ds(sid * rows_per_tile, rows_per_tile)],
                out_h.at[cid].at[pl.ds(lo + sid * rows_per_tile,
                                       rows_per_tile)])

        def put_tgt(g, vals16):
            plsc.store_scatter(
                tgt_v, [g // 8 + 0 * _iota16(),
                        (g % 8) * LANES + _iota16()], vals16)

        def fire_scatters(vb):
            ds_ = [pltpu.async_copy(vb.at[pl.ds(k * 128, 128)],
                                    acc_s.at[tgt_v.at[k]], sems, add=True)
                   for k in range(4)]
            for d in ds_:
                d.wait()

        if n_ranges == 1:
            zero_slice()
            plsc.subcore_barrier()
            chunks = _static_chunks(n_contrib, SCR)
            vbufs = (vb0, vb1)
            sems_g = (semg0, semg1)
            dload = {}

            def start_load(ci):
                off, sz = chunks[ci]
                dload[ci] = pltpu.async_copy(
                    vals_h.at[pl.ds(tbase + off, sz)],
                    vbufs[ci % 2].at[pl.ds(0, sz)], sems_g[ci % 2])

            start_load(0)
            for ci, (off, sz) in enumerate(chunks):
                if ci + 1 < len(chunks):
                    start_load(ci + 1)
                dload.pop(ci).wait()
                ng = -(-sz // LANES)
                for g in range(ng):
                    iv = _vload(idx_v, off + g * LANES)
                    rem = sz - g * LANES
                    if rem < LANES:
                        iv = jnp.where(_iota16() < rem, iv, rng_size)
                    put_tgt(g, iv)
                for g in range(ng, SCR // LANES):
                    put_tgt(g, rng_size + 0 * _iota16())
                fire_scatters(vbufs[ci % 2])
            plsc.subcore_barrier()
            dump_slice(0)
        else:
            for r in range(n_ranges):
                zero_slice()
                lo = r * rng_size

                def scan_g(g, cnt):
                    iv = _vload(idx_v, g * LANES)
                    mask = lax.shift_right_logical(iv, shift) == r
                    plsc.store_compressed(
                        bkt_v.at[pl.ds(cnt, LANES)],
                        tbase + g * LANES + _iota16(), mask=mask)
                    return cnt + jnp.sum(mask.astype(i32))

                cnt = lax.fori_loop(0, n_contrib // LANES, scan_g, 0)
                for g in range(SCR // LANES):
                    _vstore(bkt_v, cnt + g * LANES, tbase + 0 * _iota16())
                plsc.subcore_barrier()

                def sc_loop(t, _):
                    soff = t * SCR
                    dg = [pltpu.async_copy(
                        vals_h.at[bkt_v.at[pl.ds(soff + k * 128, 128)]],
                        vb0.at[pl.ds(k * 128, 128)], semg0)
                        for k in range(4)]
                    for d in dg:
                        d.wait()

                    def tgt_g(g, _):
                        gpos = _vload(bkt_v, soff + g * LANES)
                        iv = plsc.load_gather(idx_v, [gpos - tbase])
                        valid = (soff + g * LANES + _iota16()) < cnt
                        put_tgt(g, jnp.where(valid, iv - lo, rng_size))
                        return 0

                    lax.fori_loop(0, SCR // LANES, tgt_g, 0)
                    fire_scatters(vb0)
                    return 0

                trips = lax.shift_right_logical(cnt + (SCR - 1), 9)
                lax.fori_loop(0, trips, sc_loop, 0)
                plsc.subcore_barrier()
                dump_slice(lo)
                plsc.subcore_barrier()

    kern = pl.kernel(
        body,
        out_type=jax.ShapeDtypeStruct((2, m_out, EMB), f32),
        mesh=plsc.VectorSubcoreMesh(core_axis_name="c", subcore_axis_name="s"),
        compiler_params=pltpu.CompilerParams(needs_layout_passes=False,
                                             use_tc_tiling_on_sc=False),
        scratch_types=[
            pltpu.VMEM((n_contrib + 16,), i32),
            pltpu.VMEM((n_contrib + SCR,), i32),
            pltpu.VMEM((4, 128), i32),
            pltpu.VMEM((SCR, EMB), f32),
            pltpu.VMEM((SCR, EMB), f32),
            pltpu.VMEM((128, EMB), f32),
            pltpu.VMEM_SHARED((rng_size + TRASH, EMB), f32),
            pltpu.SemaphoreType.DMA,
            pltpu.SemaphoreType.DMA,
            pltpu.SemaphoreType.DMA,
        ],
    )
    return kern


_segsum_edge_k = None
_segsum_atom_k = None


def _get_segsum_edge():
    global _segsum_edge_k
    if _segsum_edge_k is None:
        _segsum_edge_k = _make_segsum(NG, R_EDGE, NRANGE_EDGE, SHIFT_EDGE)
    return _segsum_edge_k


def _get_segsum_atom():
    global _segsum_atom_k
    if _segsum_atom_k is None:
        _segsum_atom_k = _make_segsum(NE, R_ATOM, 1, 0)
    return _segsum_atom_k


def _swish(x):
    return x / (1.0 + jnp.exp(-x))


def _envelope(x):
    p = 6
    a = -(p + 1) * (p + 2) / 2.0
    b = float(p * (p + 2))
    c = -p * (p + 1) / 2.0
    x2 = x * x
    x4 = x2 * x2
    x5 = x4 * x
    env = 1.0 / x + a * x5 + b * x5 * x + c * x5 * x2
    return jnp.where(x < 1.0, env, 0.0)


NE_PAD = 163840
NG_PAD = 327680


def _env_sin(dq):
    d = jnp.sqrt(dq + 1e-12)
    x = d / CUT
    env = _envelope(x)
    return [env * jnp.sin((n + 1.0) * jnp.pi * x) for n in range(6)]


def _tc_erbf_body(dq_ref, er_ref):
    rads = _env_sin(dq_ref[...])
    s = jnp.sqrt(2.0 / CUT)
    for n in range(6):
        er_ref[n] = rads[n] * s
    z = jnp.zeros_like(rads[0])
    er_ref[6] = z
    er_ref[7] = z


def _tc_erbf(dsq_pad):
    BB = 256
    return pl.pallas_call(
        _tc_erbf_body,
        grid=(NE_PAD // 128 // BB,),
        in_specs=[pl.BlockSpec((BB, 128), lambda i: (i, 0))],
        out_specs=pl.BlockSpec((8, BB, 128), lambda i: (0, i, 0)),
        out_shape=jax.ShapeDtypeStruct((8, NE_PAD // 128, 128), f32),
    )(dsq_pad)


def _tc_sbf_body(dq_ref, ad_ref, ac_ref, sbf_ref):
    rads = _env_sin(dq_ref[...])
    adot = ad_ref[...]
    acsq = ac_ref[...]
    cosa = adot * lax.rsqrt(adot * adot + acsq + 1e-12)
    ts = [jnp.ones_like(cosa), cosa]
    for _ in range(5):
        ts.append(2.0 * cosa * ts[-1] - ts[-2])
    for n in range(6):
        for l in range(7):
            sbf_ref[n * 7 + l] = (rads[n] * ts[l]).astype(bf16)
    z = jnp.zeros_like(rads[0]).astype(bf16)
    for p in range(42, 48):
        sbf_ref[p] = z


def _tc_sbf(dqk_pad, ad_pad, ac_pad):
    BB = 256
    return pl.pallas_call(
        _tc_sbf_body,
        grid=(NG_PAD // 128 // BB,),
        in_specs=[pl.BlockSpec((BB, 128), lambda i: (i, 0))] * 3,
        out_specs=pl.BlockSpec((48, BB, 128), lambda i: (0, i, 0)),
        out_shape=jax.ShapeDtypeStruct((48, NG_PAD // 128, 128), bf16),
    )(dqk_pad, ad_pad, ac_pad)


def _tc_edge_init_body(e1_ref, e1b_ref, er_ref, werc_ref, b_ref, wr0_ref,
                       m_ref, g0_ref):
    er = er_ref[...]
    m = _swish(e1_ref[...] + e1b_ref[...]
               + _mxu(er, werc_ref[...]) + b_ref[...])
    m_ref[...] = m
    g0_ref[...] = m * _mxu(er, wr0_ref[...])


def _tc_edge_init(e1, e1b, er, werc8, bvec, wrbf0):
    BE = 2000
    return pl.pallas_call(
        _tc_edge_init_body,
        grid=(NE // BE,),
        in_specs=[
            pl.BlockSpec((BE, EMB), lambda i: (i, 0)),
            pl.BlockSpec((BE, EMB), lambda i: (i, 0)),
            pl.BlockSpec((BE, 8), lambda i: (i, 0)),
            pl.BlockSpec((8, EMB), lambda i: (0, 0)),
            pl.BlockSpec((1, EMB), lambda i: (0, 0)),
            pl.BlockSpec((8, EMB), lambda i: (0, 0)),
        ],
        out_specs=[
            pl.BlockSpec((BE, EMB), lambda i: (i, 0)),
            pl.BlockSpec((BE, EMB), lambda i: (i, 0)),
        ],
        out_shape=[
            jax.ShapeDtypeStruct((NE, EMB), f32),
            jax.ShapeDtypeStruct((NE, EMB), f32),
        ],
    )(e1, e1b, er, werc8, bvec, wrbf0)


def _tc_layer_pre_body(m_ref, er_ref, w1_ref, b1_ref, w2_ref, b2_ref,
                       wr_ref, xji_ref, q_ref):
    m = m_ref[...]
    er = er_ref[...]
    xji_ref[...] = _swish(_mxu(m, w1_ref[...]) + b1_ref[...])
    q_ref[...] = (_swish(_mxu(m, w2_ref[...]) + b2_ref[...])
                  * _mxu(er, wr_ref[...])).astype(bf16)


def _tc_layer_pre(m, er, w1, b1, w2, b2, wrbf8):
    BE = 2000
    return pl.pallas_call(
        _tc_layer_pre_body,
        grid=(NE // BE,),
        in_specs=[
            pl.BlockSpec((BE, EMB), lambda i: (i, 0)),
            pl.BlockSpec((BE, 8), lambda i: (i, 0)),
            pl.BlockSpec((EMB, EMB), lambda i: (0, 0)),
            pl.BlockSpec((1, EMB), lambda i: (0, 0)),
            pl.BlockSpec((EMB, EMB), lambda i: (0, 0)),
            pl.BlockSpec((1, EMB), lambda i: (0, 0)),
            pl.BlockSpec((8, EMB), lambda i: (0, 0)),
        ],
        out_specs=[
            pl.BlockSpec((BE, EMB), lambda i: (i, 0)),
            pl.BlockSpec((BE, EMB), lambda i: (i, 0)),
        ],
        out_shape=[
            jax.ShapeDtypeStruct((NE, EMB), f32),
            jax.ShapeDtypeStruct((NE, EMB), bf16),
        ],
    )(m, er, w1, b1, w2, b2, wrbf8)


def _tc_bilinear_body(qk_ref, sbf_ref, wb_ref, wse_ref, y_ref):
    h = _mxu(qk_ref[...], wb_ref[...])
    spb = _mxu(sbf_ref[...], wse_ref[...])
    u = spb * h
    y = u[:, 0:EMB]
    for b in range(1, NBILIN):
        y = y + u[:, b * EMB:(b + 1) * EMB]
    y_ref[...] = y


def _tc_bilinear(qk, sbf48, wbcat, wspE):
    BA = 4000
    return pl.pallas_call(
        _tc_bilinear_body,
        grid=(NG // BA,),
        in_specs=[
            pl.BlockSpec((BA, EMB), lambda i: (i, 0)),
            pl.BlockSpec((BA, 48), lambda i: (i, 0)),
            pl.BlockSpec((EMB, EMB * NBILIN), lambda i: (0, 0)),
            pl.BlockSpec((48, EMB * NBILIN), lambda i: (0, 0)),
        ],
        out_specs=pl.BlockSpec((BA, EMB), lambda i: (i, 0)),
        out_shape=jax.ShapeDtypeStruct((NG, EMB), f32),
    )(qk, sbf48, wbcat, wspE)


def _tc_layer_post_body(m_ref, xji_ref, a0_ref, a1_ref, er_ref,
                        wres_ref, bres_ref, wr_ref, mn_ref, g_ref):
    u = xji_ref[...] + a0_ref[0] + a1_ref[0]
    mn = m_ref[...] + _swish(_mxu(u, wres_ref[...]) + bres_ref[...])
    mn_ref[...] = mn
    g_ref[...] = mn * _mxu(er_ref[...], wr_ref[...])


def _tc_layer_post(m, xji, aggp, er, wres, bres, wrbf8):
    BE = 2000
    return pl.pallas_call(
        _tc_layer_post_body,
        grid=(NE // BE,),
        in_specs=[
            pl.BlockSpec((BE, EMB), lambda i: (i, 0)),
            pl.BlockSpec((BE, EMB), lambda i: (i, 0)),
            pl.BlockSpec((1, BE, EMB), lambda i: (0, i, 0)),
            pl.BlockSpec((1, BE, EMB), lambda i: (1, i, 0)),
            pl.BlockSpec((BE, 8), lambda i: (i, 0)),
            pl.BlockSpec((EMB, EMB), lambda i: (0, 0)),
            pl.BlockSpec((1, EMB), lambda i: (0, 0)),
            pl.BlockSpec((8, EMB), lambda i: (0, 0)),
        ],
        out_specs=[
            pl.BlockSpec((BE, EMB), lambda i: (i, 0)),
            pl.BlockSpec((BE, EMB), lambda i: (i, 0)),
        ],
        out_shape=[
            jax.ShapeDtypeStruct((NE, EMB), f32),
            jax.ShapeDtypeStruct((NE, EMB), f32),
        ],
    )(m, xji, aggp, aggp, er, wres, bres, wrbf8)


def _tc_atom_body(*refs):
    i = pl.program_id(0)
    out_ref = refs[-1]
    BT = refs[0].shape[1]
    rows = i * BT + lax.broadcasted_iota(i32, (BT, 1), 0)
    valid = (rows < NA).astype(f32)
    total = jnp.zeros((), f32)
    for k in range(NCONV + 1):
        tp0 = refs[2 * k][0]
        tp1 = refs[2 * k + 1][0]
        wh = refs[8 + 3 * k][...]
        bh = refs[8 + 3 * k + 1][...]
        wo = refs[8 + 3 * k + 2][...]
        s = _swish(_mxu(tp0 + tp1, wh) + bh)
        contrib = _mxu(s, wo) * valid
        total = total + jnp.sum(contrib)

    @pl.when(i == 0)
    def _():
        out_ref[...] = jnp.zeros((1, 1), f32)

    out_ref[...] += jnp.reshape(total, (1, 1))


def _tc_atom(tps, wsets):
    BT = 2528
    in_specs = []
    args = []
    for tp in tps:
        in_specs += [pl.BlockSpec((1, BT, EMB), lambda i: (0, i, 0)),
                     pl.BlockSpec((1, BT, EMB), lambda i: (1, i, 0))]
        args += [tp, tp]
    for (wh, bh, wo) in wsets:
        in_specs += [pl.BlockSpec((EMB, EMB), lambda i: (0, 0)),
                     pl.BlockSpec((1, EMB), lambda i: (0, 0)),
                     pl.BlockSpec((EMB, 1), lambda i: (0, 0))]
        args += [wh, bh, wo]
    return pl.pallas_call(
        _tc_atom_body,
        grid=(R_ATOM // BT,),
        in_specs=in_specs,
        out_specs=pl.BlockSpec((1, 1), lambda i: (0, 0)),
        out_shape=jax.ShapeDtypeStruct((1, 1), f32),
    )(*args)


def kernel(nxyz, params, nbr_list, angle_list, num_atoms, ji_idx, kj_idx):
    xs = jnp.asarray(nxyz[:, 1], f32)
    ys = jnp.asarray(nxyz[:, 2], f32)
    zs = jnp.asarray(nxyz[:, 3], f32)
    za = nxyz[:, 0].astype(i32)
    nb0 = jnp.asarray(nbr_list[:, 0], i32)
    nb1 = jnp.asarray(nbr_list[:, 1], i32)
    g0 = jnp.asarray(angle_list[:, 0], i32)
    g1 = jnp.asarray(angle_list[:, 1], i32)
    g2 = jnp.asarray(angle_list[:, 2], i32)
    ji = jnp.asarray(ji_idx, i32)
    kj = jnp.asarray(kj_idx, i32)

    W = params["emb_W"]
    a1f = (params["emb"] @ W[0:EMB]).reshape(-1)
    a2f = (params["emb"] @ W[EMB:2 * EMB]).reshape(-1)
    werc8 = jnp.pad(params["emb_Wrbf"] @ W[2 * EMB:], ((0, 2), (0, 0)))
    bvec = params["emb_b"].reshape(1, EMB)

    def pad8(w):
        return jnp.pad(w, ((0, 2), (0, 0)))

    emat = jnp.kron(jnp.eye(NBILIN, dtype=f32), jnp.ones((1, EMB), f32))
    wspE_list = []
    wb_list = []
    for l in range(NCONV):
        p = params["int"][l]
        wspE_list.append(jnp.pad(p["W_sbf"], ((0, 6), (0, 0))) @ emat)
        wb_list.append(jnp.transpose(p["W_bilin"], (1, 0, 2))
                       .reshape(EMB, NBILIN * EMB))

    dsq, adot, acsq, p1f, p2f = _sc_geom(xs, ys, zs, za, a1f, a2f,
                                         nb0, nb1, g0, g1, g2)
    eg = _get_gather(EMB, NE, dtype_f32=True)
    e1a = eg(p1f.reshape(NA_P, EMB), nb0)
    e1b = eg(p2f.reshape(NA_P, EMB), nb1)

    dsq_pad = jnp.pad(dsq, (0, NE_PAD - NE),
                      constant_values=1e6).reshape(NE_PAD // 128, 128)
    er = _tc_erbf(dsq_pad).reshape(8, NE_PAD).T
    m, g0e = _tc_edge_init(e1a, e1b, er, werc8, bvec,
                           pad8(params["out"][0]["W_rbf"]))

    dqk = _sc_gather_scalar(dsq, kj)

    def _padg(a, v):
        return jnp.pad(a, (0, NG_PAD - NG),
                       constant_values=v).reshape(NG_PAD // 128, 128)

    sbf48 = _tc_sbf(_padg(dqk, 1e6), _padg(adot, 0.0),
                    _padg(acsq, 0.0)).reshape(48, NG_PAD).T

    segsum_atom = _get_segsum_atom()
    segsum_edge = _get_segsum_edge()
    tps = [segsum_atom(g0e, nb0)]

    for l in range(NCONV):
        p = params["int"][l]
        xji, q = _tc_layer_pre(m, er, p["W1"], p["b1"].reshape(1, EMB),
                               p["W2"], p["b2"].reshape(1, EMB),
                               pad8(p["W_rbf"]))
        qk = _sc_gather_rows(q, kj)
        y = _tc_bilinear(qk, sbf48, wb_list[l], wspE_list[l])
        aggp = segsum_edge(y, ji)
        m, ge = _tc_layer_post(m, xji, aggp, er,
                               p["W_res"], p["b_res"].reshape(1, EMB),
                               pad8(params["out"][l + 1]["W_rbf"]))
        tps.append(segsum_atom(ge, nb0))

    wsets = [(po["W_h"], po["b_h"].reshape(1, EMB), po["W_out"])
             for po in params["out"]]
    total = _tc_atom(tps, wsets)
    return jnp.reshape(total, (1,))


# speedup vs baseline: 1.5549x; 1.0048x over previous
import functools

import jax
import jax.numpy as jnp
from jax import lax
from jax.experimental import pallas as pl
from jax.experimental.pallas import tpu as pltpu
from jax.experimental.pallas import tpu_sc as plsc

NA = 10000
NE = 160000
NG = 320000
EMB = 64
NRBF = 6
CUT = 5.0
NBILIN = 8
NCONV = 3

NW = 32
LANES = 16

f32 = jnp.float32
i32 = jnp.int32
bf16 = jnp.bfloat16


def _mxu(a, b):
    return jnp.dot(a.astype(bf16), b.astype(bf16), preferred_element_type=f32)

R_EDGE = 16384
SHIFT_EDGE = 14
NRANGE_EDGE = -(-NE // R_EDGE)
R_ATOM = 10112
TRASH = 16


def _iota16():
    return lax.iota(i32, LANES)


def _vload(ref, off):
    return plsc.load_gather(ref, [off + _iota16()])


def _vstore(ref, off, x, mask=None):
    plsc.store_scatter(ref, [off + _iota16()], x, mask=mask)


def _tile_chunk_range(wid, n_chunks):
    q, rem = divmod(n_chunks, NW)
    count = q + jnp.where(wid < rem, 1, 0)
    first = wid * q + jnp.minimum(wid, rem)
    return first, count


NA_P = 10016


def _sc_geom_body(xs_h, ys_h, zs_h, za_h, a1_h, a2_h, nb0_h, nb1_h,
                  g0_h, g1_h, g2_h,
                  dsq_h, adot_h, acsq_h, p1_h, p2_h,
                  xs_v, ys_v, zs_v, za_v, a1_v, a2_v,
                  eb0, eb1, dq_b, pb1, pb2,
                  gb0, gb1, gb2, ad_b, ac_b):
    wid = lax.axis_index("s") * 2 + lax.axis_index("c")
    pltpu.sync_copy(xs_h, xs_v)
    pltpu.sync_copy(ys_h, ys_v)
    pltpu.sync_copy(zs_h, zs_v)
    pltpu.sync_copy(za_h, za_v.at[pl.ds(0, NA)])
    pltpu.sync_copy(a1_h, a1_v)
    pltpu.sync_copy(a2_h, a2_v)

    APT = NA_P // NW
    abase = wid * APT

    def atom_grp(g, _):
        zraw = plsc.load_gather(za_v, [abase + g * LANES + _iota16()])
        zc = jnp.minimum(jnp.maximum(zraw, 0), 94) * EMB
        roff = g * LANES * EMB + _iota16() * EMB
        for ccol in range(EMB):
            plsc.store_scatter(pb1, [roff + ccol],
                               plsc.load_gather(a1_v, [zc + ccol]))
            plsc.store_scatter(pb2, [roff + ccol],
                               plsc.load_gather(a2_v, [zc + ccol]))
        return 0

    lax.fori_loop(0, APT // LANES + 1, atom_grp, 0)
    pltpu.sync_copy(pb1.at[pl.ds(0, APT * EMB)],
                    p1_h.at[pl.ds(abase * EMB, APT * EMB)])
    pltpu.sync_copy(pb2.at[pl.ds(0, APT * EMB)],
                    p2_h.at[pl.ds(abase * EMB, APT * EMB)])

    CE = 400
    first, count = _tile_chunk_range(wid, NE // CE)

    def edge_chunk(c, _):
        base = (first + c) * CE
        pltpu.sync_copy(nb0_h.at[pl.ds(base, CE)], eb0)
        pltpu.sync_copy(nb1_h.at[pl.ds(base, CE)], eb1)

        def grp(g, _):
            off = g * LANES
            s = _vload(eb0, off)
            t = _vload(eb1, off)
            dx = plsc.load_gather(xs_v, [s]) - plsc.load_gather(xs_v, [t])
            dy = plsc.load_gather(ys_v, [s]) - plsc.load_gather(ys_v, [t])
            dz = plsc.load_gather(zs_v, [s]) - plsc.load_gather(zs_v, [t])
            _vstore(dq_b, off, dx * dx + dy * dy + dz * dz)
            return 0

        lax.fori_loop(0, CE // LANES, grp, 0)
        pltpu.sync_copy(dq_b, dsq_h.at[pl.ds(base, CE)])
        return 0

    lax.fori_loop(0, count, edge_chunk, 0)

    CA = 512
    afirst, acount = _tile_chunk_range(wid, NG // CA)

    def ang_chunk(c, _):
        base = (afirst + c) * CA
        pltpu.sync_copy(g0_h.at[pl.ds(base, CA)], gb0)
        pltpu.sync_copy(g1_h.at[pl.ds(base, CA)], gb1)
        pltpu.sync_copy(g2_h.at[pl.ds(base, CA)], gb2)

        def grp(g, _):
            off = g * LANES
            ia = _vload(gb0, off)
            ib = _vload(gb1, off)
            ic = _vload(gb2, off)
            bx = plsc.load_gather(xs_v, [ib])
            by = plsc.load_gather(ys_v, [ib])
            bz = plsc.load_gather(zs_v, [ib])
            jx = plsc.load_gather(xs_v, [ia]) - bx
            jy = plsc.load_gather(ys_v, [ia]) - by
            jz = plsc.load_gather(zs_v, [ia]) - bz
            kx = plsc.load_gather(xs_v, [ic]) - bx
            ky = plsc.load_gather(ys_v, [ic]) - by
            kz = plsc.load_gather(zs_v, [ic]) - bz
            _vstore(ad_b, off, jx * kx + jy * ky + jz * kz)
            cx = jy * kz - jz * ky
            cy = jz * kx - jx * kz
            cz = jx * ky - jy * kx
            _vstore(ac_b, off, cx * cx + cy * cy + cz * cz)
            return 0

        lax.fori_loop(0, CA // LANES, grp, 0)
        pltpu.sync_copy(ad_b, adot_h.at[pl.ds(base, CA)])
        pltpu.sync_copy(ac_b, acsq_h.at[pl.ds(base, CA)])
        return 0

    lax.fori_loop(0, acount, ang_chunk, 0)


def _sc_geom(xs, ys, zs, za, a1f, a2f, nb0, nb1, g0, g1, g2):
    CE, CA = 400, 512
    APT = NA_P // NW
    kern = pl.kernel(
        _sc_geom_body,
        out_type=(
            jax.ShapeDtypeStruct((NE,), f32),
            jax.ShapeDtypeStruct((NG,), f32),
            jax.ShapeDtypeStruct((NG,), f32),
            jax.ShapeDtypeStruct((NA_P * EMB,), f32),
            jax.ShapeDtypeStruct((NA_P * EMB,), f32),
        ),
        mesh=plsc.VectorSubcoreMesh(core_axis_name="c", subcore_axis_name="s"),
        compiler_params=pltpu.CompilerParams(needs_layout_passes=False,
                                             use_tc_tiling_on_sc=False),
        scratch_types=[
            pltpu.VMEM((NA,), f32), pltpu.VMEM((NA,), f32),
            pltpu.VMEM((NA,), f32), pltpu.VMEM((NA_P + LANES,), i32),
            pltpu.VMEM((95 * EMB,), f32), pltpu.VMEM((95 * EMB,), f32),
            pltpu.VMEM((CE,), i32), pltpu.VMEM((CE,), i32),
            pltpu.VMEM((CE,), f32),
            pltpu.VMEM(((APT + LANES) * EMB,), f32),
            pltpu.VMEM(((APT + LANES) * EMB,), f32),
            pltpu.VMEM((CA,), i32), pltpu.VMEM((CA,), i32),
            pltpu.VMEM((CA,), i32),
            pltpu.VMEM((CA,), f32), pltpu.VMEM((CA,), f32),
        ],
    )
    return kern(xs, ys, zs, za, a1f, a2f, nb0, nb1, g0, g1, g2)


def _make_gather(width, n_out=NG, dtype_f32=False):
    SCR = 512
    n_contrib = n_out // NW

    def body(table_h, idx_h, out_h, idx_v, vb0, vb1, semg0, semg1):
        wid = lax.axis_index("s") * 2 + lax.axis_index("c")
        tbase = wid * n_contrib
        pltpu.sync_copy(idx_h.at[pl.ds(tbase, n_contrib)], idx_v)
        chunks = _static_chunks(n_contrib, SCR)
        vbufs = (vb0, vb1)
        sems_g = (semg0, semg1)
        dload = {}

        def start_load(ci):
            off, sz = chunks[ci]
            dload[ci] = [pltpu.async_copy(
                table_h.at[idx_v.at[pl.ds(off + koff, ksz)]],
                vbufs[ci % 2].at[pl.ds(koff, ksz)], sems_g[ci % 2])
                for koff, ksz in _static_chunks(sz, 128)]

        start_load(0)
        for ci, (off, sz) in enumerate(chunks):
            if ci + 1 < len(chunks):
                start_load(ci + 1)
            for d in dload.pop(ci):
                d.wait()
            pltpu.sync_copy(vbufs[ci % 2].at[pl.ds(0, sz)],
                            out_h.at[pl.ds(tbase + off, sz)])

    out_shape = (n_out, EMB) if width else (n_out,)
    buf_shape = (SCR, EMB) if width else (SCR,)
    dt = f32 if (dtype_f32 or not width) else bf16
    kern = pl.kernel(
        body,
        out_type=jax.ShapeDtypeStruct(out_shape, dt),
        mesh=plsc.VectorSubcoreMesh(core_axis_name="c", subcore_axis_name="s"),
        compiler_params=pltpu.CompilerParams(needs_layout_passes=False,
                                             use_tc_tiling_on_sc=False),
        scratch_types=[
            pltpu.VMEM((n_contrib,), i32),
            pltpu.VMEM(buf_shape, dt),
            pltpu.VMEM(buf_shape, dt),
            pltpu.SemaphoreType.DMA,
            pltpu.SemaphoreType.DMA,
        ],
    )
    return kern


_gather_cache = {}


def _get_gather(width, n_out=NG, dtype_f32=False):
    key = (width, n_out, dtype_f32)
    if key not in _gather_cache:
        _gather_cache[key] = _make_gather(width, n_out, dtype_f32)
    return _gather_cache[key]


def _sc_gather_rows(table, idx):
    return _get_gather(EMB)(table, idx)


def _sc_gather_scalar(table, idx):
    return _get_gather(0)(table, idx)


def _static_chunks(total, c):
    out_list = []
    off = 0
    while off < total:
        sz = min(c, total - off)
        out_list.append((off, sz))
        off += sz
    return out_list


def _make_segsum(n_in, rng_size, n_ranges, shift):
    SCR = 512
    n_contrib = n_in // NW
    rows_per_tile = rng_size // LANES
    m_out = rng_size * n_ranges

    def body(vals_h, idx_h, out_h, idx_v, bkt_v, tgt_v, vb0, vb1, zbuf_v,
             acc_s, semg0, semg1, sems):
        cid = lax.axis_index("c")
        sid = lax.axis_index("s")
        wid = sid * 2 + cid
        tbase = wid * n_contrib
        pltpu.sync_copy(idx_h.at[pl.ds(tbase, n_contrib)],
                        idx_v.at[pl.ds(0, n_contrib)])

        def zinit(i, _):
            zbuf_v[i // 4, pl.ds((i % 4) * LANES, LANES)] = (
                jnp.zeros((LANES,), f32))
            return 0

        lax.fori_loop(0, 128 * 4, zinit, 0)

        def zero_slice():
            for (zoff, zsz) in _static_chunks(rows_per_tile, 128):
                pltpu.sync_copy(
                    zbuf_v.at[pl.ds(0, zsz)],
                    acc_s.at[pl.ds(sid * rows_per_tile + zoff, zsz)])

        def dump_slice(lo):
            pltpu.sync_copy(
                acc_s.at[pl.ds(sid * rows_per_tile, rows_per_tile)],
                out_h.at[cid].at[pl.ds(lo + sid * rows_per_tile,
                                       rows_per_tile)])

        def put_tgt(g, vals16):
            plsc.store_scatter(
                tgt_v, [g // 8 + 0 * _iota16(),
                        (g % 8) * LANES + _iota16()], vals16)

        def fire_scatters(vb):
            ds_ = [pltpu.async_copy(vb.at[pl.ds(k * 128, 128)],
                                    acc_s.at[tgt_v.at[k]], sems, add=True)
                   for k in range(4)]
            for d in ds_:
                d.wait()

        if n_ranges == 1:
            zero_slice()
            plsc.subcore_barrier()
            chunks = _static_chunks(n_contrib, SCR)
            vbufs = (vb0, vb1)
            sems_g = (semg0, semg1)
            dload = {}

            def start_load(ci):
                off, sz = chunks[ci]
                dload[ci] = pltpu.async_copy(
                    vals_h.at[pl.ds(tbase + off, sz)],
                    vbufs[ci % 2].at[pl.ds(0, sz)], sems_g[ci % 2])

            start_load(0)
            for ci, (off, sz) in enumerate(chunks):
                if ci + 1 < len(chunks):
                    start_load(ci + 1)
                dload.pop(ci).wait()
                ng = -(-sz // LANES)
                for g in range(ng):
                    iv = _vload(idx_v, off + g * LANES)
                    rem = sz - g * LANES
                    if rem < LANES:
                        iv = jnp.where(_iota16() < rem, iv, rng_size)
                    put_tgt(g, iv)
                for g in range(ng, SCR // LANES):
                    put_tgt(g, rng_size + 0 * _iota16())
                fire_scatters(vbufs[ci % 2])
            plsc.subcore_barrier()
            dump_slice(0)
        else:
            for r in range(n_ranges):
                zero_slice()
                lo = r * rng_size

                def scan_g(g, cnt):
                    iv = _vload(idx_v, g * LANES)
                    mask = lax.shift_right_logical(iv, shift) == r
                    plsc.store_compressed(
                        bkt_v.at[pl.ds(cnt, LANES)],
                        tbase + g * LANES + _iota16(), mask=mask)
                    return cnt + jnp.sum(mask.astype(i32))

                cnt = lax.fori_loop(0, n_contrib // LANES, scan_g, 0)
                for g in range(SCR // LANES):
                    _vstore(bkt_v, cnt + g * LANES, tbase + 0 * _iota16())
                plsc.subcore_barrier()

                def sc_loop(t, _):
                    soff = t * SCR
                    dg = [pltpu.async_copy(
                        vals_h.at[bkt_v.at[pl.ds(soff + k * 128, 128)]],
                        vb0.at[pl.ds(k * 128, 128)], semg0)
                        for k in range(4)]

                    def tgt_g(g, _):
                        gpos = _vload(bkt_v, soff + g * LANES)
                        iv = plsc.load_gather(idx_v, [gpos - tbase])
                        valid = (soff + g * LANES + _iota16()) < cnt
                        put_tgt(g, jnp.where(valid, iv - lo, rng_size))
                        return 0

                    lax.fori_loop(0, SCR // LANES, tgt_g, 0)
                    for d in dg:
                        d.wait()
                    fire_scatters(vb0)
                    return 0

                trips = lax.shift_right_logical(cnt + (SCR - 1), 9)
                lax.fori_loop(0, trips, sc_loop, 0)
                plsc.subcore_barrier()
                dump_slice(lo)
                plsc.subcore_barrier()

    kern = pl.kernel(
        body,
        out_type=jax.ShapeDtypeStruct((2, m_out, EMB), f32),
        mesh=plsc.VectorSubcoreMesh(core_axis_name="c", subcore_axis_name="s"),
        compiler_params=pltpu.CompilerParams(needs_layout_passes=False,
                                             use_tc_tiling_on_sc=False),
        scratch_types=[
            pltpu.VMEM((n_contrib + 16,), i32),
            pltpu.VMEM((n_contrib + SCR,), i32),
            pltpu.VMEM((4, 128), i32),
            pltpu.VMEM((SCR, EMB), f32),
            pltpu.VMEM((SCR, EMB), f32),
            pltpu.VMEM((128, EMB), f32),
            pltpu.VMEM_SHARED((rng_size + TRASH, EMB), f32),
            pltpu.SemaphoreType.DMA,
            pltpu.SemaphoreType.DMA,
            pltpu.SemaphoreType.DMA,
        ],
    )
    return kern


_segsum_edge_k = None
_segsum_atom_k = None


def _get_segsum_edge():
    global _segsum_edge_k
    if _segsum_edge_k is None:
        _segsum_edge_k = _make_segsum(NG, R_EDGE, NRANGE_EDGE, SHIFT_EDGE)
    return _segsum_edge_k


def _get_segsum_atom():
    global _segsum_atom_k
    if _segsum_atom_k is None:
        _segsum_atom_k = _make_segsum(NE, R_ATOM, 1, 0)
    return _segsum_atom_k


def _swish(x):
    return x / (1.0 + jnp.exp(-x))


def _envelope(x):
    p = 6
    a = -(p + 1) * (p + 2) / 2.0
    b = float(p * (p + 2))
    c = -p * (p + 1) / 2.0
    x2 = x * x
    x4 = x2 * x2
    x5 = x4 * x
    env = 1.0 / x + a * x5 + b * x5 * x + c * x5 * x2
    return jnp.where(x < 1.0, env, 0.0)


NE_PAD = 163840
NG_PAD = 327680


def _env_sin(dq):
    d = jnp.sqrt(dq + 1e-12)
    x = d / CUT
    env = _envelope(x)
    return [env * jnp.sin((n + 1.0) * jnp.pi * x) for n in range(6)]


def _tc_erbf_body(dq_ref, er_ref):
    rads = _env_sin(dq_ref[...])
    s = jnp.sqrt(2.0 / CUT)
    for n in range(6):
        er_ref[n] = rads[n] * s
    z = jnp.zeros_like(rads[0])
    er_ref[6] = z
    er_ref[7] = z


def _tc_erbf(dsq_pad):
    BB = 256
    return pl.pallas_call(
        _tc_erbf_body,
        grid=(NE_PAD // 128 // BB,),
        in_specs=[pl.BlockSpec((BB, 128), lambda i: (i, 0))],
        out_specs=pl.BlockSpec((8, BB, 128), lambda i: (0, i, 0)),
        out_shape=jax.ShapeDtypeStruct((8, NE_PAD // 128, 128), f32),
    )(dsq_pad)


def _tc_sbf_body(dq_ref, ad_ref, ac_ref, sbf_ref):
    rads = _env_sin(dq_ref[...])
    adot = ad_ref[...]
    acsq = ac_ref[...]
    cosa = adot * lax.rsqrt(adot * adot + acsq + 1e-12)
    ts = [jnp.ones_like(cosa), cosa]
    for _ in range(5):
        ts.append(2.0 * cosa * ts[-1] - ts[-2])
    for n in range(6):
        for l in range(7):
            sbf_ref[n * 7 + l] = (rads[n] * ts[l]).astype(bf16)
    z = jnp.zeros_like(rads[0]).astype(bf16)
    for p in range(42, 48):
        sbf_ref[p] = z


def _tc_sbf(dqk_pad, ad_pad, ac_pad):
    BB = 256
    return pl.pallas_call(
        _tc_sbf_body,
        grid=(NG_PAD // 128 // BB,),
        in_specs=[pl.BlockSpec((BB, 128), lambda i: (i, 0))] * 3,
        out_specs=pl.BlockSpec((48, BB, 128), lambda i: (0, i, 0)),
        out_shape=jax.ShapeDtypeStruct((48, NG_PAD // 128, 128), bf16),
    )(dqk_pad, ad_pad, ac_pad)


def _tc_edge_init_body(e1_ref, e1b_ref, er_ref, werc_ref, b_ref, wr0_ref,
                       m_ref, g0_ref):
    er = er_ref[...]
    m = _swish(e1_ref[...] + e1b_ref[...]
               + _mxu(er, werc_ref[...]) + b_ref[...])
    m_ref[...] = m
    g0_ref[...] = m * _mxu(er, wr0_ref[...])


def _tc_edge_init(e1, e1b, er, werc8, bvec, wrbf0):
    BE = 2000
    return pl.pallas_call(
        _tc_edge_init_body,
        grid=(NE // BE,),
        in_specs=[
            pl.BlockSpec((BE, EMB), lambda i: (i, 0)),
            pl.BlockSpec((BE, EMB), lambda i: (i, 0)),
            pl.BlockSpec((BE, 8), lambda i: (i, 0)),
            pl.BlockSpec((8, EMB), lambda i: (0, 0)),
            pl.BlockSpec((1, EMB), lambda i: (0, 0)),
            pl.BlockSpec((8, EMB), lambda i: (0, 0)),
        ],
        out_specs=[
            pl.BlockSpec((BE, EMB), lambda i: (i, 0)),
            pl.BlockSpec((BE, EMB), lambda i: (i, 0)),
        ],
        out_shape=[
            jax.ShapeDtypeStruct((NE, EMB), f32),
            jax.ShapeDtypeStruct((NE, EMB), f32),
        ],
    )(e1, e1b, er, werc8, bvec, wrbf0)


def _tc_layer_pre_body(m_ref, er_ref, w1_ref, b1_ref, w2_ref, b2_ref,
                       wr_ref, xji_ref, q_ref):
    m = m_ref[...]
    er = er_ref[...]
    xji_ref[...] = _swish(_mxu(m, w1_ref[...]) + b1_ref[...])
    q_ref[...] = (_swish(_mxu(m, w2_ref[...]) + b2_ref[...])
                  * _mxu(er, wr_ref[...])).astype(bf16)


def _tc_layer_pre(m, er, w1, b1, w2, b2, wrbf8):
    BE = 2000
    return pl.pallas_call(
        _tc_layer_pre_body,
        grid=(NE // BE,),
        in_specs=[
            pl.BlockSpec((BE, EMB), lambda i: (i, 0)),
            pl.BlockSpec((BE, 8), lambda i: (i, 0)),
            pl.BlockSpec((EMB, EMB), lambda i: (0, 0)),
            pl.BlockSpec((1, EMB), lambda i: (0, 0)),
            pl.BlockSpec((EMB, EMB), lambda i: (0, 0)),
            pl.BlockSpec((1, EMB), lambda i: (0, 0)),
            pl.BlockSpec((8, EMB), lambda i: (0, 0)),
        ],
        out_specs=[
            pl.BlockSpec((BE, EMB), lambda i: (i, 0)),
            pl.BlockSpec((BE, EMB), lambda i: (i, 0)),
        ],
        out_shape=[
            jax.ShapeDtypeStruct((NE, EMB), f32),
            jax.ShapeDtypeStruct((NE, EMB), bf16),
        ],
    )(m, er, w1, b1, w2, b2, wrbf8)


def _tc_bilinear_body(qk_ref, sbf_ref, wb_ref, wse_ref, y_ref):
    h = _mxu(qk_ref[...], wb_ref[...])
    spb = _mxu(sbf_ref[...], wse_ref[...])
    u = spb * h
    y = u[:, 0:EMB]
    for b in range(1, NBILIN):
        y = y + u[:, b * EMB:(b + 1) * EMB]
    y_ref[...] = y


def _tc_bilinear(qk, sbf48, wbcat, wspE):
    BA = 4000
    return pl.pallas_call(
        _tc_bilinear_body,
        grid=(NG // BA,),
        in_specs=[
            pl.BlockSpec((BA, EMB), lambda i: (i, 0)),
            pl.BlockSpec((BA, 48), lambda i: (i, 0)),
            pl.BlockSpec((EMB, EMB * NBILIN), lambda i: (0, 0)),
            pl.BlockSpec((48, EMB * NBILIN), lambda i: (0, 0)),
        ],
        out_specs=pl.BlockSpec((BA, EMB), lambda i: (i, 0)),
        out_shape=jax.ShapeDtypeStruct((NG, EMB), f32),
    )(qk, sbf48, wbcat, wspE)


def _tc_layer_post_body(m_ref, xji_ref, a0_ref, a1_ref, er_ref,
                        wres_ref, bres_ref, wr_ref, mn_ref, g_ref):
    u = xji_ref[...] + a0_ref[0] + a1_ref[0]
    mn = m_ref[...] + _swish(_mxu(u, wres_ref[...]) + bres_ref[...])
    mn_ref[...] = mn
    g_ref[...] = mn * _mxu(er_ref[...], wr_ref[...])


def _tc_layer_post(m, xji, aggp, er, wres, bres, wrbf8):
    BE = 2000
    return pl.pallas_call(
        _tc_layer_post_body,
        grid=(NE // BE,),
        in_specs=[
            pl.BlockSpec((BE, EMB), lambda i: (i, 0)),
            pl.BlockSpec((BE, EMB), lambda i: (i, 0)),
            pl.BlockSpec((1, BE, EMB), lambda i: (0, i, 0)),
            pl.BlockSpec((1, BE, EMB), lambda i: (1, i, 0)),
            pl.BlockSpec((BE, 8), lambda i: (i, 0)),
            pl.BlockSpec((EMB, EMB), lambda i: (0, 0)),
            pl.BlockSpec((1, EMB), lambda i: (0, 0)),
            pl.BlockSpec((8, EMB), lambda i: (0, 0)),
        ],
        out_specs=[
            pl.BlockSpec((BE, EMB), lambda i: (i, 0)),
            pl.BlockSpec((BE, EMB), lambda i: (i, 0)),
        ],
        out_shape=[
            jax.ShapeDtypeStruct((NE, EMB), f32),
            jax.ShapeDtypeStruct((NE, EMB), f32),
        ],
    )(m, xji, aggp, aggp, er, wres, bres, wrbf8)


def _tc_atom_body(*refs):
    i = pl.program_id(0)
    out_ref = refs[-1]
    BT = refs[0].shape[1]
    rows = i * BT + lax.broadcasted_iota(i32, (BT, 1), 0)
    valid = (rows < NA).astype(f32)
    total = jnp.zeros((), f32)
    for k in range(NCONV + 1):
        tp0 = refs[2 * k][0]
        tp1 = refs[2 * k + 1][0]
        wh = refs[8 + 3 * k][...]
        bh = refs[8 + 3 * k + 1][...]
        wo = refs[8 + 3 * k + 2][...]
        s = _swish(_mxu(tp0 + tp1, wh) + bh)
        contrib = _mxu(s, wo) * valid
        total = total + jnp.sum(contrib)

    @pl.when(i == 0)
    def _():
        out_ref[...] = jnp.zeros((1, 1), f32)

    out_ref[...] += jnp.reshape(total, (1, 1))


def _tc_atom(tps, wsets):
    BT = 2528
    in_specs = []
    args = []
    for tp in tps:
        in_specs += [pl.BlockSpec((1, BT, EMB), lambda i: (0, i, 0)),
                     pl.BlockSpec((1, BT, EMB), lambda i: (1, i, 0))]
        args += [tp, tp]
    for (wh, bh, wo) in wsets:
        in_specs += [pl.BlockSpec((EMB, EMB), lambda i: (0, 0)),
                     pl.BlockSpec((1, EMB), lambda i: (0, 0)),
                     pl.BlockSpec((EMB, 1), lambda i: (0, 0))]
        args += [wh, bh, wo]
    return pl.pallas_call(
        _tc_atom_body,
        grid=(R_ATOM // BT,),
        in_specs=in_specs,
        out_specs=pl.BlockSpec((1, 1), lambda i: (0, 0)),
        out_shape=jax.ShapeDtypeStruct((1, 1), f32),
    )(*args)


def kernel(nxyz, params, nbr_list, angle_list, num_atoms, ji_idx, kj_idx):
    xs = jnp.asarray(nxyz[:, 1], f32)
    ys = jnp.asarray(nxyz[:, 2], f32)
    zs = jnp.asarray(nxyz[:, 3], f32)
    za = nxyz[:, 0].astype(i32)
    nb0 = jnp.asarray(nbr_list[:, 0], i32)
    nb1 = jnp.asarray(nbr_list[:, 1], i32)
    g0 = jnp.asarray(angle_list[:, 0], i32)
    g1 = jnp.asarray(angle_list[:, 1], i32)
    g2 = jnp.asarray(angle_list[:, 2], i32)
    ji = jnp.asarray(ji_idx, i32)
    kj = jnp.asarray(kj_idx, i32)

    W = params["emb_W"]
    a1f = (params["emb"] @ W[0:EMB]).reshape(-1)
    a2f = (params["emb"] @ W[EMB:2 * EMB]).reshape(-1)
    werc8 = jnp.pad(params["emb_Wrbf"] @ W[2 * EMB:], ((0, 2), (0, 0)))
    bvec = params["emb_b"].reshape(1, EMB)

    def pad8(w):
        return jnp.pad(w, ((0, 2), (0, 0)))

    emat = jnp.kron(jnp.eye(NBILIN, dtype=f32), jnp.ones((1, EMB), f32))
    wspE_list = []
    wb_list = []
    for l in range(NCONV):
        p = params["int"][l]
        wspE_list.append(jnp.pad(p["W_sbf"], ((0, 6), (0, 0))) @ emat)
        wb_list.append(jnp.transpose(p["W_bilin"], (1, 0, 2))
                       .reshape(EMB, NBILIN * EMB))

    dsq, adot, acsq, p1f, p2f = _sc_geom(xs, ys, zs, za, a1f, a2f,
                                         nb0, nb1, g0, g1, g2)
    eg = _get_gather(EMB, NE, dtype_f32=True)
    e1a = eg(p1f.reshape(NA_P, EMB), nb0)
    e1b = eg(p2f.reshape(NA_P, EMB), nb1)

    dsq_pad = jnp.pad(dsq, (0, NE_PAD - NE),
                      constant_values=1e6).reshape(NE_PAD // 128, 128)
    er = _tc_erbf(dsq_pad).reshape(8, NE_PAD).T
    m, g0e = _tc_edge_init(e1a, e1b, er, werc8, bvec,
                           pad8(params["out"][0]["W_rbf"]))

    dqk = _sc_gather_scalar(dsq, kj)

    def _padg(a, v):
        return jnp.pad(a, (0, NG_PAD - NG),
                       constant_values=v).reshape(NG_PAD // 128, 128)

    sbf48 = _tc_sbf(_padg(dqk, 1e6), _padg(adot, 0.0),
                    _padg(acsq, 0.0)).reshape(48, NG_PAD).T

    segsum_atom = _get_segsum_atom()
    segsum_edge = _get_segsum_edge()
    tps = [segsum_atom(g0e, nb0)]

    for l in range(NCONV):
        p = params["int"][l]
        xji, q = _tc_layer_pre(m, er, p["W1"], p["b1"].reshape(1, EMB),
                               p["W2"], p["b2"].reshape(1, EMB),
                               pad8(p["W_rbf"]))
        qk = _sc_gather_rows(q, kj)
        y = _tc_bilinear(qk, sbf48, wb_list[l], wspE_list[l])
        aggp = segsum_edge(y, ji)
        m, ge = _tc_layer_post(m, xji, aggp, er,
                               p["W_res"], p["b_res"].reshape(1, EMB),
                               pad8(params["out"][l + 1]["W_rbf"]))
        tps.append(segsum_atom(ge, nb0))

    wsets = [(po["W_h"], po["b_h"].reshape(1, EMB), po["W_out"])
             for po in params["out"]]
    total = _tc_atom(tps, wsets)
    return jnp.reshape(total, (1,))
